# baseline (speedup 1.0000x reference)
import functools

import jax
import jax.numpy as jnp
import numpy as np
from jax import lax
from jax.experimental import pallas as pl
from jax.experimental.pallas import tpu as pltpu
from jax.experimental.pallas import tpu_sc as plsc

H = 128
NRBF = 32
NL = 4
ANF = 16
RNF = 21
NV = 8
CUT = 8.0
NA = 10000
NRES = 2000
N = NA + NRES
E = 192000
NW = 32
K = 240
EB = 1920
NB = 1000
NPAD = 12032

_MESH = plsc.VectorSubcoreMesh(core_axis_name="c", subcore_axis_name="s")


def _silu(x):
    return x * jax.nn.sigmoid(x)


def _ln(h, g, b):
    mu = jnp.mean(h, axis=-1, keepdims=True)
    va = jnp.mean((h - mu) ** 2, axis=-1, keepdims=True)
    return (h - mu) / jnp.sqrt(va + 1e-5) * g + b


def _tc_encoder(xf, p, in_wh, in_wt, in_b, wm0, t, tw, tb, brows):
    nrows, din = xf.shape
    half = H // 2
    freq = jnp.exp(jnp.arange(half, dtype=jnp.float32)
                   * (-np.log(10000.0) / (half - 1)))[None, :]

    def body(x_ref, w1, b1, g1, bb1, w2, b2, g2, bb2, inw, inwt, inb, wm,
             t_ref, fr_ref, tw_ref, tb_ref, x0_ref, xw_ref):
        x = x_ref[...]
        h = _silu(_ln(x @ w1[...] + b1[...], g1[...], bb1[...]))
        h = _ln(h @ w2[...] + b2[...], g2[...], bb2[...])
        te_arg = t_ref[...] * fr_ref[...]
        te = (jnp.sum(jnp.sin(te_arg) * tw_ref[0:1, :])
              + jnp.sum(jnp.cos(te_arg) * tw_ref[1:2, :]) + tb_ref[0, 0])
        x0 = h @ inw[...] + te * inwt[...] + inb[...]
        x0_ref[...] = x0
        xw_ref[...] = x0 @ wm[...]

    full = lambda a, b: pl.BlockSpec((a, b), lambda i: (0, 0))
    return pl.pallas_call(
        body,
        grid=(nrows // brows,),
        in_specs=[
            pl.BlockSpec((brows, din), lambda i: (i, 0)),
            full(din, half), full(1, half), full(1, half), full(1, half),
            full(half, H), full(1, H), full(1, H), full(1, H),
            full(H, H), full(1, H), full(1, H), full(H, H),
            full(1, 1), full(1, half), full(2, half), full(1, 1),
        ],
        out_specs=[
            pl.BlockSpec((brows, H), lambda i: (i, 0)),
            pl.BlockSpec((brows, H), lambda i: (i, 0)),
        ],
        out_shape=[
            jax.ShapeDtypeStruct((nrows, H), jnp.float32),
            jax.ShapeDtypeStruct((nrows, H), jnp.float32),
        ],
    )(xf, p['w1'], p['b1'][None], p['g1'][None], p['bb1'][None],
      p['w2'], p['b2'][None], p['g2'][None], p['bb2'][None],
      in_wh, in_wt, in_b, wm0, t[:, None], freq, tw, tb)


def _sc_pos_diff(pos128, src, dst):
    @functools.partial(
        pl.kernel,
        mesh=_MESH,
        out_type=jax.ShapeDtypeStruct((E, 128), jnp.float32),
        scratch_types=[
            pltpu.VMEM((K,), jnp.int32),
            pltpu.VMEM((K,), jnp.int32),
            pltpu.VMEM((K, 128), jnp.float32),
            pltpu.VMEM((K, 128), jnp.float32),
            pltpu.SemaphoreType.DMA,
            pltpu.SemaphoreType.DMA,
        ],
    )
    def k(pos_hbm, src_hbm, dst_hbm, out_hbm, si_v, di_v, a_v, b_v, s1, s2):
        wid = lax.axis_index("s") * 2 + lax.axis_index("c")
        ept = E // NW
        nslab = ept // K

        def body(i, _):
            base = wid * ept + i * K
            pltpu.sync_copy(src_hbm.at[pl.ds(base, K)], si_v)
            pltpu.sync_copy(dst_hbm.at[pl.ds(base, K)], di_v)
            ca = pltpu.async_copy(pos_hbm.at[si_v], a_v, s1)
            cb = pltpu.async_copy(pos_hbm.at[di_v], b_v, s2)
            ca.wait()
            cb.wait()

            def row(r, _):
                b_v[r, pl.ds(0, 16)] = b_v[r, pl.ds(0, 16)] - a_v[r, pl.ds(0, 16)]
                return 0

            lax.fori_loop(0, K, row, 0, unroll=4)
            pltpu.sync_copy(b_v, out_hbm.at[pl.ds(base, K)])
            return 0

        lax.fori_loop(0, nslab, body, 0)

    return k(pos128, src, dst)


def _tc_edge_geom(diff16, etype, etype_emb):
    means = jnp.linspace(float(np.exp(-CUT)), 1.0, NRBF)[None, :]
    beta = float(((2.0 / NRBF) * (1.0 - np.exp(-CUT))) ** -2)

    def body(diff_ref, et_ref, emb_ref, means_ref, efb_ref, wts_ref):
        dif = diff_ref[...]
        dx, dy, dz = dif[:, 0:1], dif[:, 1:2], dif[:, 2:3]
        d = jnp.sqrt(dx * dx + dy * dy + dz * dz + 1e-12)
        ux, uy, uz = dx / d, dy / d, dz / d
        C = jnp.where(d < CUT, 0.5 * (jnp.cos(jnp.pi * d / CUT) + 1.0), 0.0)
        rbf = jnp.exp(-beta * (jnp.exp(-d) - means_ref[...]) ** 2)
        et = et_ref[...][:, 0]
        emb = emb_ref[...]
        emb_sel = (jnp.where((et == 0)[:, None], emb[0][None, :], 0.0)
                   + jnp.where((et == 1)[:, None], emb[1][None, :], 0.0)
                   + jnp.where((et == 2)[:, None], emb[2][None, :], 0.0))
        efb_ref[...] = rbf * C + emb_sel
        z = jnp.zeros_like(C)
        wts_ref[...] = jnp.concatenate([
            C, C * ux, C * uy, C * uz,
            C * ux * uy, C * uy * uz, C * uz * ux,
            C * (ux * ux - uy * uy), C * (3.0 * uz * uz - 1.0),
            z, z, z, z, z, z, z], axis=1)

    return pl.pallas_call(
        body,
        grid=(E // EB,),
        in_specs=[
            pl.BlockSpec((EB, 128), lambda i: (i, 0)),
            pl.BlockSpec((EB, 1), lambda i: (i, 0)),
            pl.BlockSpec((3, NRBF), lambda i: (0, 0)),
            pl.BlockSpec((1, NRBF), lambda i: (0, 0)),
        ],
        out_specs=[
            pl.BlockSpec((EB, NRBF), lambda i: (i, 0)),
            pl.BlockSpec((EB, 16), lambda i: (i, 0)),
        ],
        out_shape=[
            jax.ShapeDtypeStruct((E, NRBF), jnp.float32),
            jax.ShapeDtypeStruct((E, 16), jnp.float32),
        ],
    )(diff16, etype[:, None], etype_emb, means)


def _tc_edge_dense(efb, we, be, wm, bm):
    def body(efb_ref, we_r, be_r, wm_r, bm_r, out_ref):
        ef = _silu(efb_ref[...] @ we_r[...] + be_r[...])
        out_ref[...] = ef @ wm_r[...] + bm_r[...]

    return pl.pallas_call(
        body,
        grid=(E // EB,),
        in_specs=[
            pl.BlockSpec((EB, NRBF), lambda i: (i, 0)),
            pl.BlockSpec((NRBF, H), lambda i: (0, 0)),
            pl.BlockSpec((1, H), lambda i: (0, 0)),
            pl.BlockSpec((H, H), lambda i: (0, 0)),
            pl.BlockSpec((1, H), lambda i: (0, 0)),
        ],
        out_specs=pl.BlockSpec((EB, H), lambda i: (i, 0)),
        out_shape=jax.ShapeDtypeStruct((E, H), jnp.float32),
    )(efb, we, be[None], wm, bm[None])


def _sc_gather_silu(xw, efw, src):
    @functools.partial(
        pl.kernel,
        mesh=_MESH,
        out_type=jax.ShapeDtypeStruct((E // 2, H), jnp.int32),
        scratch_types=[
            pltpu.VMEM((K,), jnp.int32),
            pltpu.VMEM((K, H), jnp.float32),
            pltpu.VMEM((K, H), jnp.float32),
            pltpu.VMEM((K // 2, H), jnp.int32),
            pltpu.SemaphoreType.DMA,
        ],
    )
    def k(xw_hbm, efw_hbm, src_hbm, out_hbm, idx_v, g_v, e_v, o_v, sem):
        wid = lax.axis_index("s") * 2 + lax.axis_index("c")
        ept = E // NW
        nslab = ept // K
        himask = jnp.int32(-65536)

        def body(i, _):
            base = wid * ept + i * K
            pltpu.sync_copy(src_hbm.at[pl.ds(base, K)], idx_v)
            pltpu.async_copy(xw_hbm.at[idx_v], g_v, sem).wait()
            pltpu.sync_copy(efw_hbm.at[pl.ds(base, K)], e_v)

            def pair(rp, _):
                for half in range(2):
                    r = rp * 2 + half
                    for j in range(H // 32):
                        p0 = (g_v[r, pl.ds(j * 32, 16)]
                              + e_v[r, pl.ds(j * 32, 16)])
                        p1 = (g_v[r, pl.ds(j * 32 + 16, 16)]
                              + e_v[r, pl.ds(j * 32 + 16, 16)])
                        s0 = p0 / (1.0 + jnp.exp(-p0))
                        s1 = p1 / (1.0 + jnp.exp(-p1))
                        u0 = lax.bitcast_convert_type(s0, jnp.int32)
                        u1 = lax.bitcast_convert_type(s1, jnp.int32)
                        u0 = u0 + (32767 + ((u0 >> 16) & 1))
                        u1 = u1 + (32767 + ((u1 >> 16) & 1))
                        o_v[rp, pl.ds(half * 64 + j * 16, 16)] = (
                            ((u0 >> 16) & 65535) | (u1 & himask))
                return 0

            lax.fori_loop(0, K // 2, pair, 0)
            pltpu.sync_copy(
                o_v, out_hbm.at[pl.ds(pl.multiple_of(base // 2, 8), K // 2)])
            return 0

        lax.fori_loop(0, nslab, body, 0)

    return k(xw, efw, src)


def _sc_scatter9(sil, wts_flat, dst):
    KS = 96

    @functools.partial(
        pl.kernel,
        mesh=_MESH,
        out_type=jax.ShapeDtypeStruct((9 * NPAD, H), jnp.float32),
        scratch_types=[
            pltpu.VMEM((KS,), jnp.int32),
            pltpu.VMEM((KS,), jnp.float32),
            pltpu.VMEM((KS // 2, H), jnp.int32),
            pltpu.VMEM((KS, H), jnp.float32),
            pltpu.VMEM((16, H), jnp.float32),
            pltpu.VMEM_SHARED((NPAD, H), jnp.float32),
        ],
    )
    def k(sil_hbm, wts_hbm, dst_hbm, out_hbm, didx_v, w_v, m_v, s_v, z_v, acc):
        cid = lax.axis_index("c")
        sid = lax.axis_index("s")
        ept = E // 16
        nslab = ept // KS
        trows = NPAD // 16

        for r in range(16):
            for j in range(H // 16):
                z_v[r, pl.ds(j * 16, 16)] = jnp.zeros((16,), jnp.float32)

        def one_pass(p, _):
            ci = p * 2 + cid

            @pl.when(ci < 9)
            def _():
                def zbody(i, _):
                    pltpu.sync_copy(z_v, acc.at[pl.ds(sid * trows + i * 16, 16)])
                    return 0

                lax.fori_loop(0, trows // 16, zbody, 0)
                plsc.subcore_barrier()

                himask = jnp.int32(-65536)

                def body(i, _):
                    base = sid * ept + i * KS
                    pltpu.sync_copy(dst_hbm.at[pl.ds(base, KS)], didx_v)
                    pltpu.sync_copy(
                        sil_hbm.at[pl.ds(pl.multiple_of(base // 2, 8),
                                         KS // 2)], m_v)
                    pltpu.sync_copy(wts_hbm.at[pl.ds(ci * E + base, KS)], w_v)

                    def grp(g, _):
                        w16 = w_v[pl.ds(g * 16, 16)]
                        for l in range(16):
                            spl = w16.at[jnp.full((16,), l, jnp.int32)].get(
                                mode='promise_in_bounds')
                            r = g * 16 + l
                            for j in range(H // 32):
                                pk = m_v[g * 8 + l // 2,
                                         pl.ds((l % 2) * 64 + j * 16, 16)]
                                a = lax.bitcast_convert_type(
                                    pk << 16, jnp.float32)
                                b = lax.bitcast_convert_type(
                                    pk & himask, jnp.float32)
                                s_v[r, pl.ds(j * 32, 16)] = a * spl
                                s_v[r, pl.ds(j * 32 + 16, 16)] = b * spl
                        return 0

                    lax.fori_loop(0, KS // 16, grp, 0)
                    pltpu.sync_copy(s_v, acc.at[didx_v], add=True)
                    return 0

                lax.fori_loop(0, nslab, body, 0)
                plsc.subcore_barrier()
                pltpu.sync_copy(acc.at[pl.ds(sid * trows, trows)],
                                out_hbm.at[pl.ds(ci * NPAD + sid * trows, trows)])
                plsc.subcore_barrier()

            return 0

        lax.fori_loop(0, 5, one_pass, 0)

    return k(sil, wts_flat, dst)


def _tc_node_update(x, dv0, wu, bu, wm_next):
    def body(x_ref, dv_ref, wu_r, bu_r, wm_r, xo_ref, xw_ref):
        y = x_ref[...] + dv_ref[...]
        y = y + _silu(y @ wu_r[...] + bu_r[...])
        xo_ref[...] = y
        xw_ref[...] = y @ wm_r[...]

    return pl.pallas_call(
        body,
        grid=(N // NB,),
        in_specs=[
            pl.BlockSpec((NB, H), lambda i: (i, 0)),
            pl.BlockSpec((NB, H), lambda i: (i, 0)),
            pl.BlockSpec((H, H), lambda i: (0, 0)),
            pl.BlockSpec((1, H), lambda i: (0, 0)),
            pl.BlockSpec((H, H), lambda i: (0, 0)),
        ],
        out_specs=[
            pl.BlockSpec((NB, H), lambda i: (i, 0)),
            pl.BlockSpec((NB, H), lambda i: (i, 0)),
        ],
        out_shape=[
            jax.ShapeDtypeStruct((N, H), jnp.float32),
            jax.ShapeDtypeStruct((N, H), jnp.float32),
        ],
    )(x, dv0, wu, bu[None], wm_next)


def _geb_blk(x, v, wv1, wv2, u1w, u1b, u2w, u2b, dout, scalar_act):
    nv, nb, din = v.shape
    vec1 = (v.reshape(nv * nb, din) @ wv1).reshape(nv, nb, din)
    vec1n = jnp.sqrt(jnp.sum(vec1 * vec1, axis=0) + 1e-12)
    vec2 = (v.reshape(nv * nb, din) @ wv2).reshape(nv, nb, wv2.shape[1])
    h = jnp.concatenate([x, vec1n], axis=-1)
    h = _silu(h @ u1w + u1b)
    h = h @ u2w + u2b
    xo, gate = h[:, :dout], h[:, dout:]
    vo = vec2 * gate[None, :, :]
    if scalar_act:
        xo = _silu(xo)
    return xo, vo


def _tc_head(x, dvv, cw, fw):
    def body(x_ref, d1, d2, d3, d4,
             c_wv1, c_wv2, c_u1w, c_u1b, c_u2w, c_u2b,
             c2_wv1, c2_wv2, c2_u1w, c2_u1b, c2_u2w, c2_u2b,
             sn_w1, sn_b1, sn_w2, sn_b2, l1_w, l1_b,
             l2_w1, l2_b1, l2_w2, l2_b2, comb,
             f_wv1, f_wv2, f_u1w, f_u1b, f_u2w, f_u2b,
             f2_wv1, f2_wv2, f2_u1w, f2_u1b, f2_u2w, f2_u2b,
             vel_ref, xf_ref):
        x = x_ref[...]
        v = d1[...] + d2[...] + d3[...] + d4[...]
        fsel0 = lambda r: r[...][0]
        xc, vc = _geb_blk(x, v, c_wv1[...], c_wv2[...], c_u1w[...],
                          c_u1b[...], c_u2w[...], c_u2b[...], H // 2, True)
        xc, vc = _geb_blk(xc, vc, c2_wv1[...], c2_wv2[...], c2_u1w[...],
                          c2_u1b[...], c2_u2w[...], c2_u2b[...], H // 4, True)
        l1w = l1_w[...]
        l1v = jnp.concatenate(
            [jnp.sum(vc[kk] * l1w, axis=-1, keepdims=True) for kk in range(3)],
            axis=1) + l1_b[...]
        vl2 = vc[3:8]
        l2m = jnp.sqrt(jnp.sum(vl2 * vl2, axis=0) + 1e-12)
        l2mod = jnp.tanh(_silu(l2m @ l2_w1[...] + l2_b1[...]) @ l2_w2[...] + l2_b2[...])
        mag = jax.nn.sigmoid(_silu(xc @ sn_w1[...] + sn_b1[...]) @ sn_w2[...] + sn_b2[...])
        c0 = comb[0, 0]
        c1 = comb[0, 1]
        e0 = jnp.exp(c0 - jnp.maximum(c0, c1))
        e1 = jnp.exp(c1 - jnp.maximum(c0, c1))
        w0 = e0 / (e0 + e1)
        w1 = e1 / (e0 + e1)
        vel = mag * ((w0 + w1 * l2mod) * l1v)
        vel_ref[...] = jnp.concatenate([vel, jnp.zeros((vel.shape[0], 5), jnp.float32)], axis=1)
        xa, va = _geb_blk(x, v, fsel0(f_wv1), fsel0(f_wv2), fsel0(f_u1w),
                          fsel0(f_u1b), fsel0(f_u2w), fsel0(f_u2b), H // 2, True)
        xa, _ = _geb_blk(xa, va, fsel0(f2_wv1), fsel0(f2_wv2), fsel0(f2_u1w),
                         fsel0(f2_u1b), fsel0(f2_u2w), fsel0(f2_u2b), RNF, False)
        xf_ref[...] = xa

    full = lambda *s: pl.BlockSpec(s, lambda i: (0,) * len(s))
    fsel = lambda *s: pl.BlockSpec((1,) + s[1:],
                                   lambda i: (i // 10,) + (0,) * (len(s) - 1))
    dspec = pl.BlockSpec((NV, NB, H), lambda i: (0, i, 0))
    vel, xf = pl.pallas_call(
        body,
        grid=(N // NB,),
        in_specs=[
            pl.BlockSpec((NB, H), lambda i: (i, 0)),
            dspec, dspec, dspec, dspec,
            full(H, H), full(H, H // 2), full(2 * H, H), full(1, H),
            full(H, H), full(1, H),
            full(H // 2, H // 2), full(H // 2, H // 4), full(H, H // 2),
            full(1, H // 2), full(H // 2, H // 2), full(1, H // 2),
            full(H // 4, H // 8), full(1, H // 8), full(H // 8, 1), full(1, 1),
            full(1, H // 4), full(1, 1),
            full(H // 4, H // 8), full(1, H // 8), full(H // 8, 1), full(1, 1),
            full(1, 2),
            fsel(2, H, H), fsel(2, H, H // 2), fsel(2, 2 * H, H), fsel(2, 1, H),
            fsel(2, H, H), fsel(2, 1, H),
            fsel(2, H // 2, H // 2), fsel(2, H // 2, RNF), fsel(2, H, H // 2),
            fsel(2, 1, H // 2), fsel(2, H // 2, 2 * RNF), fsel(2, 1, 2 * RNF),
        ],
        out_specs=[
            pl.BlockSpec((NB, 8), lambda i: (i, 0)),
            pl.BlockSpec((NB, RNF), lambda i: (i, 0)),
        ],
        out_shape=[
            jax.ShapeDtypeStruct((N, 8), jnp.float32),
            jax.ShapeDtypeStruct((N, RNF), jnp.float32),
        ],
    )(x, dvv[0], dvv[1], dvv[2], dvv[3], *cw, *fw)
    return vel, xf


def _pad_cols(a, w):
    return jnp.pad(a, ((0, 0), (0, w - a.shape[1])))


def kernel(xh_atoms, xh_residues, t, mask_atoms, mask_residues, edge_index, edge_types, params):
    src = edge_index[0].astype(jnp.int32)
    dst = edge_index[1].astype(jnp.int32)
    etype = edge_types.astype(jnp.int32)
    lp = params['layers']
    in_w = params['in_w']
    tw = jnp.stack([params['time_w'][:H // 2, 0], params['time_w'][H // 2:, 0]])
    tb = params['time_b'][None]

    x0a, xwa = _tc_encoder(xh_atoms[:, 3:], params['atom_enc'], in_w[:H],
                           in_w[H:H + 1], params['in_b'][None], lp[0]['wm'],
                           t, tw, tb, 2000)
    x0r, xwr = _tc_encoder(xh_residues[:, 3:], params['res_enc'], in_w[:H],
                           in_w[H:H + 1], params['in_b'][None], lp[0]['wm'],
                           t, tw, tb, 2000)
    x = jnp.concatenate([x0a, x0r], axis=0)
    xw = jnp.concatenate([xwa, xwr], axis=0)

    pos128 = jnp.zeros((N, 128), jnp.float32).at[:, :3].set(
        jnp.concatenate([xh_atoms[:, :3], xh_residues[:, :3]], axis=0))
    diff = _sc_pos_diff(pos128, src, dst)
    efb, wts_e = _tc_edge_geom(diff, etype, params['etype_emb'])
    wts_flat = wts_e.T.reshape(-1)

    dvvs = []
    for li in range(NL):
        p = lp[li]
        efw = _tc_edge_dense(efb, p['we'], p['be'], p['wm'], p['bm'])
        sil = _sc_gather_silu(xw, efw, src)
        dv9 = _sc_scatter9(sil, wts_flat, dst).reshape(9, NPAD, H)
        dvvs.append(dv9[1:9, :N])
        wm_next = lp[li + 1]['wm'] if li + 1 < NL else p['wu']
        x, xw = _tc_node_update(x, dv9[0, :N], p['wu'], p['bu'], wm_next)

    cp = params['coord']
    cw = []
    for g, dout in ((cp['g1'], H // 2), (cp['g2'], H // 4)):
        cw += [g['wv1'], g['wv2'], g['u1w'], g['u1b'][None], g['u2w'], g['u2b'][None]]
    cw += [cp['sn_w1'], cp['sn_b1'][None], cp['sn_w2'], cp['sn_b2'][None],
           cp['l1_w'].T, cp['l1_b'][None],
           cp['l2_w1'], cp['l2_b1'][None], cp['l2_w2'], cp['l2_b2'][None],
           cp['comb'][None]]

    fa, fr = params['feat_a'], params['feat_r']
    fw = []
    for key in ('wv1', 'wv2', 'u1w', 'u1b', 'u2w', 'u2b'):
        a, r = fa['g1'][key], fr['g1'][key]
        if a.ndim == 1:
            a, r = a[None], r[None]
        fw.append(jnp.stack([a, r]))
    a2, r2 = fa['g2'], fr['g2']
    u2w_a = jnp.concatenate([_pad_cols(a2['u2w'][:, :ANF], RNF),
                             _pad_cols(a2['u2w'][:, ANF:], RNF)], axis=1)
    u2b_a = jnp.concatenate([jnp.pad(a2['u2b'][:ANF], (0, RNF - ANF)),
                             jnp.pad(a2['u2b'][ANF:], (0, RNF - ANF))])
    for key, aw, rw in (('wv1', a2['wv1'], r2['wv1']),
                        ('wv2', _pad_cols(a2['wv2'], RNF), r2['wv2']),
                        ('u1w', a2['u1w'], r2['u1w']),
                        ('u1b', a2['u1b'][None], r2['u1b'][None]),
                        ('u2w', u2w_a, r2['u2w']),
                        ('u2b', u2b_a[None], r2['u2b'][None])):
        fw.append(jnp.stack([aw, rw]))

    vel, xf = _tc_head(x, dvvs, cw, fw)
    out_a = jnp.concatenate([vel[:NA, :3], xf[:NA, :ANF]], axis=-1)
    out_r = jnp.concatenate([vel[NA:, :3], xf[NA:, :RNF]], axis=-1)
    return (out_a, out_r)


# speedup vs baseline: 1.8625x; 1.8625x over previous
import functools

import jax
import jax.numpy as jnp
import numpy as np
from jax import lax
from jax.experimental import pallas as pl
from jax.experimental.pallas import tpu as pltpu
from jax.experimental.pallas import tpu_sc as plsc

H = 128
NRBF = 32
NL = 4
ANF = 16
RNF = 21
NV = 8
CUT = 8.0
NA = 10000
NRES = 2000
N = NA + NRES
E = 192000
NW = 32
K = 240
EB = 1920
NB = 1000
NPAD = 12032

_MESH = plsc.VectorSubcoreMesh(core_axis_name="c", subcore_axis_name="s")


def _silu(x):
    return x * jax.nn.sigmoid(x)


def _ln(h, g, b):
    mu = jnp.mean(h, axis=-1, keepdims=True)
    va = jnp.mean((h - mu) ** 2, axis=-1, keepdims=True)
    return (h - mu) / jnp.sqrt(va + 1e-5) * g + b


def _tc_encoder(xf, p, in_wh, in_wt, in_b, wm0, t, tw, tb, brows):
    nrows, din = xf.shape
    half = H // 2
    freq = jnp.exp(jnp.arange(half, dtype=jnp.float32)
                   * (-np.log(10000.0) / (half - 1)))[None, :]

    def body(x_ref, w1, b1, g1, bb1, w2, b2, g2, bb2, inw, inwt, inb, wm,
             t_ref, fr_ref, tw_ref, tb_ref, x0_ref, xw_ref):
        x = x_ref[...]
        h = _silu(_ln(x @ w1[...] + b1[...], g1[...], bb1[...]))
        h = _ln(h @ w2[...] + b2[...], g2[...], bb2[...])
        te_arg = t_ref[...] * fr_ref[...]
        te = (jnp.sum(jnp.sin(te_arg) * tw_ref[0:1, :])
              + jnp.sum(jnp.cos(te_arg) * tw_ref[1:2, :]) + tb_ref[0, 0])
        x0 = h @ inw[...] + te * inwt[...] + inb[...]
        x0_ref[...] = x0
        xw_ref[...] = x0 @ wm[...]

    full = lambda a, b: pl.BlockSpec((a, b), lambda i: (0, 0))
    return pl.pallas_call(
        body,
        grid=(nrows // brows,),
        in_specs=[
            pl.BlockSpec((brows, din), lambda i: (i, 0)),
            full(din, half), full(1, half), full(1, half), full(1, half),
            full(half, H), full(1, H), full(1, H), full(1, H),
            full(H, H), full(1, H), full(1, H), full(H, H),
            full(1, 1), full(1, half), full(2, half), full(1, 1),
        ],
        out_specs=[
            pl.BlockSpec((brows, H), lambda i: (i, 0)),
            pl.BlockSpec((brows, H), lambda i: (i, 0)),
        ],
        out_shape=[
            jax.ShapeDtypeStruct((nrows, H), jnp.float32),
            jax.ShapeDtypeStruct((nrows, H), jnp.float32),
        ],
    )(xf, p['w1'], p['b1'][None], p['g1'][None], p['bb1'][None],
      p['w2'], p['b2'][None], p['g2'][None], p['bb2'][None],
      in_wh, in_wt, in_b, wm0, t[:, None], freq, tw, tb)


def _sc_pos_diff(pos128, src, dst):
    @functools.partial(
        pl.kernel,
        mesh=_MESH,
        out_type=jax.ShapeDtypeStruct((E, 128), jnp.float32),
        scratch_types=[
            pltpu.VMEM((K,), jnp.int32),
            pltpu.VMEM((K,), jnp.int32),
            pltpu.VMEM((K, 128), jnp.float32),
            pltpu.VMEM((K, 128), jnp.float32),
            pltpu.SemaphoreType.DMA,
            pltpu.SemaphoreType.DMA,
        ],
    )
    def k(pos_hbm, src_hbm, dst_hbm, out_hbm, si_v, di_v, a_v, b_v, s1, s2):
        wid = lax.axis_index("s") * 2 + lax.axis_index("c")
        ept = E // NW
        nslab = ept // K

        def body(i, _):
            base = wid * ept + i * K
            pltpu.sync_copy(src_hbm.at[pl.ds(base, K)], si_v)
            pltpu.sync_copy(dst_hbm.at[pl.ds(base, K)], di_v)
            ca = pltpu.async_copy(pos_hbm.at[si_v], a_v, s1)
            cb = pltpu.async_copy(pos_hbm.at[di_v], b_v, s2)
            ca.wait()
            cb.wait()

            def row(r, _):
                b_v[r, pl.ds(0, 16)] = b_v[r, pl.ds(0, 16)] - a_v[r, pl.ds(0, 16)]
                return 0

            lax.fori_loop(0, K, row, 0, unroll=4)
            pltpu.sync_copy(b_v, out_hbm.at[pl.ds(base, K)])
            return 0

        lax.fori_loop(0, nslab, body, 0)

    return k(pos128, src, dst)


def _tc_edge_geom(diff16, etype, etype_emb):
    means = jnp.linspace(float(np.exp(-CUT)), 1.0, NRBF)[None, :]
    beta = float(((2.0 / NRBF) * (1.0 - np.exp(-CUT))) ** -2)

    def body(diff_ref, et_ref, emb_ref, means_ref, efb_ref, wts_ref):
        dif = diff_ref[...]
        dx, dy, dz = dif[:, 0:1], dif[:, 1:2], dif[:, 2:3]
        d = jnp.sqrt(dx * dx + dy * dy + dz * dz + 1e-12)
        ux, uy, uz = dx / d, dy / d, dz / d
        C = jnp.where(d < CUT, 0.5 * (jnp.cos(jnp.pi * d / CUT) + 1.0), 0.0)
        rbf = jnp.exp(-beta * (jnp.exp(-d) - means_ref[...]) ** 2)
        et = et_ref[...][:, 0]
        emb = emb_ref[...]
        emb_sel = (jnp.where((et == 0)[:, None], emb[0][None, :], 0.0)
                   + jnp.where((et == 1)[:, None], emb[1][None, :], 0.0)
                   + jnp.where((et == 2)[:, None], emb[2][None, :], 0.0))
        efb_ref[...] = rbf * C + emb_sel
        z = jnp.zeros_like(C)
        wts_ref[...] = jnp.concatenate([
            C, C * ux, C * uy, C * uz,
            C * ux * uy, C * uy * uz, C * uz * ux,
            C * (ux * ux - uy * uy), C * (3.0 * uz * uz - 1.0),
            z, z, z, z, z, z, z], axis=1)

    return pl.pallas_call(
        body,
        grid=(E // EB,),
        in_specs=[
            pl.BlockSpec((EB, 128), lambda i: (i, 0)),
            pl.BlockSpec((EB, 1), lambda i: (i, 0)),
            pl.BlockSpec((3, NRBF), lambda i: (0, 0)),
            pl.BlockSpec((1, NRBF), lambda i: (0, 0)),
        ],
        out_specs=[
            pl.BlockSpec((EB, NRBF), lambda i: (i, 0)),
            pl.BlockSpec((EB, 16), lambda i: (i, 0)),
        ],
        out_shape=[
            jax.ShapeDtypeStruct((E, NRBF), jnp.float32),
            jax.ShapeDtypeStruct((E, 16), jnp.float32),
        ],
    )(diff16, etype[:, None], etype_emb, means)


def _tc_edge_dense(efb, we, be, wm, bm):
    def body(efb_ref, we_r, be_r, wm_r, bm_r, out_ref):
        ef = _silu(efb_ref[...] @ we_r[...] + be_r[...])
        out_ref[...] = ef @ wm_r[...] + bm_r[...]

    return pl.pallas_call(
        body,
        grid=(E // EB,),
        in_specs=[
            pl.BlockSpec((EB, NRBF), lambda i: (i, 0)),
            pl.BlockSpec((NRBF, H), lambda i: (0, 0)),
            pl.BlockSpec((1, H), lambda i: (0, 0)),
            pl.BlockSpec((H, H), lambda i: (0, 0)),
            pl.BlockSpec((1, H), lambda i: (0, 0)),
        ],
        out_specs=pl.BlockSpec((EB, H), lambda i: (i, 0)),
        out_shape=jax.ShapeDtypeStruct((E, H), jnp.float32),
    )(efb, we, be[None], wm, bm[None])


def _sc_gather_silu(xw, efw, src):
    KG = 120

    @functools.partial(
        pl.kernel,
        mesh=_MESH,
        out_type=jax.ShapeDtypeStruct((E, H), jnp.float32),
        scratch_types=[
            pltpu.VMEM((KG,), jnp.int32),
            pltpu.VMEM((KG,), jnp.int32),
            pltpu.VMEM((KG, H), jnp.float32),
            pltpu.VMEM((KG, H), jnp.float32),
            pltpu.VMEM((KG, H), jnp.float32),
            pltpu.VMEM((KG, H), jnp.float32),
            pltpu.SemaphoreType.DMA,
            pltpu.SemaphoreType.DMA,
        ],
    )
    def k(xw_hbm, efw_hbm, src_hbm, out_hbm,
          i0_v, i1_v, g0_v, g1_v, e0_v, e1_v, s0, s1):
        wid = lax.axis_index("s") * 2 + lax.axis_index("c")
        ept = E // NW
        npair = ept // (2 * KG)

        def silu_rows(g_v, e_v):
            def row(r, _):
                for j in range(H // 16):
                    p = g_v[r, pl.ds(j * 16, 16)] + e_v[r, pl.ds(j * 16, 16)]
                    e_v[r, pl.ds(j * 16, 16)] = p / (1.0 + jnp.exp(-p))
                return 0

            lax.fori_loop(0, KG, row, 0)

        def body(i, _):
            b0 = wid * ept + i * 2 * KG
            b1 = b0 + KG
            pltpu.sync_copy(src_hbm.at[pl.ds(b0, KG)], i0_v)
            ca = pltpu.async_copy(xw_hbm.at[i0_v], g0_v, s0)
            pltpu.sync_copy(src_hbm.at[pl.ds(b1, KG)], i1_v)
            cb = pltpu.async_copy(xw_hbm.at[i1_v], g1_v, s1)
            pltpu.sync_copy(efw_hbm.at[pl.ds(b0, KG)], e0_v)
            ca.wait()
            silu_rows(g0_v, e0_v)
            pltpu.sync_copy(efw_hbm.at[pl.ds(b1, KG)], e1_v)
            pltpu.sync_copy(e0_v, out_hbm.at[pl.ds(b0, KG)])
            cb.wait()
            silu_rows(g1_v, e1_v)
            pltpu.sync_copy(e1_v, out_hbm.at[pl.ds(b1, KG)])
            return 0

        lax.fori_loop(0, npair, body, 0)

    return k(xw, efw, src)


def _sc_scatter9(sil, wts_flat, dst):
    KS = 160

    @functools.partial(
        pl.kernel,
        mesh=_MESH,
        out_type=jax.ShapeDtypeStruct((9 * NPAD, H), jnp.float32),
        scratch_types=[
            pltpu.VMEM((KS,), jnp.int32),
            pltpu.VMEM((KS,), jnp.float32),
            pltpu.VMEM((KS, H), jnp.float32),
            pltpu.VMEM((16, H), jnp.float32),
            pltpu.VMEM_SHARED((NPAD, H), jnp.float32),
        ],
    )
    def k(sil_hbm, wts_hbm, dst_hbm, out_hbm, didx_v, w_v, m_v, z_v, acc):
        cid = lax.axis_index("c")
        sid = lax.axis_index("s")
        ept = E // 16
        nslab = ept // KS
        trows = NPAD // 16

        for r in range(16):
            for j in range(H // 16):
                z_v[r, pl.ds(j * 16, 16)] = jnp.zeros((16,), jnp.float32)

        def one_pass(p, _):
            ci = p * 2 + cid

            @pl.when(ci < 9)
            def _():
                def zbody(i, _):
                    pltpu.sync_copy(z_v, acc.at[pl.ds(sid * trows + i * 16, 16)])
                    return 0

                lax.fori_loop(0, trows // 16, zbody, 0)
                plsc.subcore_barrier()

                def body(i, _):
                    base = sid * ept + i * KS
                    pltpu.sync_copy(dst_hbm.at[pl.ds(base, KS)], didx_v)
                    pltpu.sync_copy(sil_hbm.at[pl.ds(base, KS)], m_v)
                    pltpu.sync_copy(wts_hbm.at[pl.ds(ci * E + base, KS)], w_v)

                    def grp(g, _):
                        w16 = w_v[pl.ds(g * 16, 16)]
                        for l in range(16):
                            spl = w16.at[jnp.full((16,), l, jnp.int32)].get(
                                mode='promise_in_bounds')
                            for j in range(H // 16):
                                m_v[g * 16 + l, pl.ds(j * 16, 16)] = (
                                    m_v[g * 16 + l, pl.ds(j * 16, 16)] * spl)
                        return 0

                    lax.fori_loop(0, KS // 16, grp, 0)
                    pltpu.sync_copy(m_v, acc.at[didx_v], add=True)
                    return 0

                lax.fori_loop(0, nslab, body, 0)
                plsc.subcore_barrier()
                pltpu.sync_copy(acc.at[pl.ds(sid * trows, trows)],
                                out_hbm.at[pl.ds(ci * NPAD + sid * trows, trows)])
                plsc.subcore_barrier()

            return 0

        lax.fori_loop(0, 5, one_pass, 0)

    return k(sil, wts_flat, dst)


def _tc_node_update(x, dv0, wu, bu, wm_next):
    def body(x_ref, dv_ref, wu_r, bu_r, wm_r, xo_ref, xw_ref):
        y = x_ref[...] + dv_ref[...]
        y = y + _silu(y @ wu_r[...] + bu_r[...])
        xo_ref[...] = y
        xw_ref[...] = y @ wm_r[...]

    return pl.pallas_call(
        body,
        grid=(N // NB,),
        in_specs=[
            pl.BlockSpec((NB, H), lambda i: (i, 0)),
            pl.BlockSpec((NB, H), lambda i: (i, 0)),
            pl.BlockSpec((H, H), lambda i: (0, 0)),
            pl.BlockSpec((1, H), lambda i: (0, 0)),
            pl.BlockSpec((H, H), lambda i: (0, 0)),
        ],
        out_specs=[
            pl.BlockSpec((NB, H), lambda i: (i, 0)),
            pl.BlockSpec((NB, H), lambda i: (i, 0)),
        ],
        out_shape=[
            jax.ShapeDtypeStruct((N, H), jnp.float32),
            jax.ShapeDtypeStruct((N, H), jnp.float32),
        ],
    )(x, dv0, wu, bu[None], wm_next)


def _geb_blk(x, v, wv1, wv2, u1w, u1b, u2w, u2b, dout, scalar_act):
    nv, nb, din = v.shape
    vec1 = (v.reshape(nv * nb, din) @ wv1).reshape(nv, nb, din)
    vec1n = jnp.sqrt(jnp.sum(vec1 * vec1, axis=0) + 1e-12)
    vec2 = (v.reshape(nv * nb, din) @ wv2).reshape(nv, nb, wv2.shape[1])
    h = jnp.concatenate([x, vec1n], axis=-1)
    h = _silu(h @ u1w + u1b)
    h = h @ u2w + u2b
    xo, gate = h[:, :dout], h[:, dout:]
    vo = vec2 * gate[None, :, :]
    if scalar_act:
        xo = _silu(xo)
    return xo, vo


def _tc_head(x, dvv, cw, fw):
    def body(x_ref, d1, d2, d3, d4,
             c_wv1, c_wv2, c_u1w, c_u1b, c_u2w, c_u2b,
             c2_wv1, c2_wv2, c2_u1w, c2_u1b, c2_u2w, c2_u2b,
             sn_w1, sn_b1, sn_w2, sn_b2, l1_w, l1_b,
             l2_w1, l2_b1, l2_w2, l2_b2, comb,
             f_wv1, f_wv2, f_u1w, f_u1b, f_u2w, f_u2b,
             f2_wv1, f2_wv2, f2_u1w, f2_u1b, f2_u2w, f2_u2b,
             vel_ref, xf_ref):
        x = x_ref[...]
        v = d1[...] + d2[...] + d3[...] + d4[...]
        fsel0 = lambda r: r[...][0]
        xc, vc = _geb_blk(x, v, c_wv1[...], c_wv2[...], c_u1w[...],
                          c_u1b[...], c_u2w[...], c_u2b[...], H // 2, True)
        xc, vc = _geb_blk(xc, vc, c2_wv1[...], c2_wv2[...], c2_u1w[...],
                          c2_u1b[...], c2_u2w[...], c2_u2b[...], H // 4, True)
        l1w = l1_w[...]
        l1v = jnp.concatenate(
            [jnp.sum(vc[kk] * l1w, axis=-1, keepdims=True) for kk in range(3)],
            axis=1) + l1_b[...]
        vl2 = vc[3:8]
        l2m = jnp.sqrt(jnp.sum(vl2 * vl2, axis=0) + 1e-12)
        l2mod = jnp.tanh(_silu(l2m @ l2_w1[...] + l2_b1[...]) @ l2_w2[...] + l2_b2[...])
        mag = jax.nn.sigmoid(_silu(xc @ sn_w1[...] + sn_b1[...]) @ sn_w2[...] + sn_b2[...])
        c0 = comb[0, 0]
        c1 = comb[0, 1]
        e0 = jnp.exp(c0 - jnp.maximum(c0, c1))
        e1 = jnp.exp(c1 - jnp.maximum(c0, c1))
        w0 = e0 / (e0 + e1)
        w1 = e1 / (e0 + e1)
        vel = mag * ((w0 + w1 * l2mod) * l1v)
        vel_ref[...] = jnp.concatenate([vel, jnp.zeros((vel.shape[0], 5), jnp.float32)], axis=1)
        xa, va = _geb_blk(x, v, fsel0(f_wv1), fsel0(f_wv2), fsel0(f_u1w),
                          fsel0(f_u1b), fsel0(f_u2w), fsel0(f_u2b), H // 2, True)
        xa, _ = _geb_blk(xa, va, fsel0(f2_wv1), fsel0(f2_wv2), fsel0(f2_u1w),
                         fsel0(f2_u1b), fsel0(f2_u2w), fsel0(f2_u2b), RNF, False)
        xf_ref[...] = xa

    full = lambda *s: pl.BlockSpec(s, lambda i: (0,) * len(s))
    fsel = lambda *s: pl.BlockSpec((1,) + s[1:],
                                   lambda i: (i // 10,) + (0,) * (len(s) - 1))
    dspec = pl.BlockSpec((NV, NB, H), lambda i: (0, i, 0))
    vel, xf = pl.pallas_call(
        body,
        grid=(N // NB,),
        in_specs=[
            pl.BlockSpec((NB, H), lambda i: (i, 0)),
            dspec, dspec, dspec, dspec,
            full(H, H), full(H, H // 2), full(2 * H, H), full(1, H),
            full(H, H), full(1, H),
            full(H // 2, H // 2), full(H // 2, H // 4), full(H, H // 2),
            full(1, H // 2), full(H // 2, H // 2), full(1, H // 2),
            full(H // 4, H // 8), full(1, H // 8), full(H // 8, 1), full(1, 1),
            full(1, H // 4), full(1, 1),
            full(H // 4, H // 8), full(1, H // 8), full(H // 8, 1), full(1, 1),
            full(1, 2),
            fsel(2, H, H), fsel(2, H, H // 2), fsel(2, 2 * H, H), fsel(2, 1, H),
            fsel(2, H, H), fsel(2, 1, H),
            fsel(2, H // 2, H // 2), fsel(2, H // 2, RNF), fsel(2, H, H // 2),
            fsel(2, 1, H // 2), fsel(2, H // 2, 2 * RNF), fsel(2, 1, 2 * RNF),
        ],
        out_specs=[
            pl.BlockSpec((NB, 8), lambda i: (i, 0)),
            pl.BlockSpec((NB, RNF), lambda i: (i, 0)),
        ],
        out_shape=[
            jax.ShapeDtypeStruct((N, 8), jnp.float32),
            jax.ShapeDtypeStruct((N, RNF), jnp.float32),
        ],
    )(x, dvv[0], dvv[1], dvv[2], dvv[3], *cw, *fw)
    return vel, xf


def _pad_cols(a, w):
    return jnp.pad(a, ((0, 0), (0, w - a.shape[1])))


def kernel(xh_atoms, xh_residues, t, mask_atoms, mask_residues, edge_index, edge_types, params):
    src = edge_index[0].astype(jnp.int32)
    dst = edge_index[1].astype(jnp.int32)
    etype = edge_types.astype(jnp.int32)
    lp = params['layers']
    in_w = params['in_w']
    tw = jnp.stack([params['time_w'][:H // 2, 0], params['time_w'][H // 2:, 0]])
    tb = params['time_b'][None]

    x0a, xwa = _tc_encoder(xh_atoms[:, 3:], params['atom_enc'], in_w[:H],
                           in_w[H:H + 1], params['in_b'][None], lp[0]['wm'],
                           t, tw, tb, 2000)
    x0r, xwr = _tc_encoder(xh_residues[:, 3:], params['res_enc'], in_w[:H],
                           in_w[H:H + 1], params['in_b'][None], lp[0]['wm'],
                           t, tw, tb, 2000)
    x = jnp.concatenate([x0a, x0r], axis=0)
    xw = jnp.concatenate([xwa, xwr], axis=0)

    pos128 = jnp.zeros((N, 128), jnp.float32).at[:, :3].set(
        jnp.concatenate([xh_atoms[:, :3], xh_residues[:, :3]], axis=0))
    diff = _sc_pos_diff(pos128, src, dst)
    efb, wts_e = _tc_edge_geom(diff, etype, params['etype_emb'])
    wts_flat = wts_e.T.reshape(-1)

    dvvs = []
    for li in range(NL):
        p = lp[li]
        efw = _tc_edge_dense(efb, p['we'], p['be'], p['wm'], p['bm'])
        sil = _sc_gather_silu(xw, efw, src)
        dv9 = _sc_scatter9(sil, wts_flat, dst).reshape(9, NPAD, H)
        dvvs.append(dv9[1:9, :N])
        wm_next = lp[li + 1]['wm'] if li + 1 < NL else p['wu']
        x, xw = _tc_node_update(x, dv9[0, :N], p['wu'], p['bu'], wm_next)

    cp = params['coord']
    cw = []
    for g, dout in ((cp['g1'], H // 2), (cp['g2'], H // 4)):
        cw += [g['wv1'], g['wv2'], g['u1w'], g['u1b'][None], g['u2w'], g['u2b'][None]]
    cw += [cp['sn_w1'], cp['sn_b1'][None], cp['sn_w2'], cp['sn_b2'][None],
           cp['l1_w'].T, cp['l1_b'][None],
           cp['l2_w1'], cp['l2_b1'][None], cp['l2_w2'], cp['l2_b2'][None],
           cp['comb'][None]]

    fa, fr = params['feat_a'], params['feat_r']
    fw = []
    for key in ('wv1', 'wv2', 'u1w', 'u1b', 'u2w', 'u2b'):
        a, r = fa['g1'][key], fr['g1'][key]
        if a.ndim == 1:
            a, r = a[None], r[None]
        fw.append(jnp.stack([a, r]))
    a2, r2 = fa['g2'], fr['g2']
    u2w_a = jnp.concatenate([_pad_cols(a2['u2w'][:, :ANF], RNF),
                             _pad_cols(a2['u2w'][:, ANF:], RNF)], axis=1)
    u2b_a = jnp.concatenate([jnp.pad(a2['u2b'][:ANF], (0, RNF - ANF)),
                             jnp.pad(a2['u2b'][ANF:], (0, RNF - ANF))])
    for key, aw, rw in (('wv1', a2['wv1'], r2['wv1']),
                        ('wv2', _pad_cols(a2['wv2'], RNF), r2['wv2']),
                        ('u1w', a2['u1w'], r2['u1w']),
                        ('u1b', a2['u1b'][None], r2['u1b'][None]),
                        ('u2w', u2w_a, r2['u2w']),
                        ('u2b', u2b_a[None], r2['u2b'][None])):
        fw.append(jnp.stack([aw, rw]))

    vel, xf = _tc_head(x, dvvs, cw, fw)
    out_a = jnp.concatenate([vel[:NA, :3], xf[:NA, :ANF]], axis=-1)
    out_r = jnp.concatenate([vel[NA:, :3], xf[NA:, :RNF]], axis=-1)
    return (out_a, out_r)


# speedup vs baseline: 1.8701x; 1.0041x over previous
import functools

import jax
import jax.numpy as jnp
import numpy as np
from jax import lax
from jax.experimental import pallas as pl
from jax.experimental.pallas import tpu as pltpu
from jax.experimental.pallas import tpu_sc as plsc

H = 128
NRBF = 32
NL = 4
ANF = 16
RNF = 21
NV = 8
CUT = 8.0
NA = 10000
NRES = 2000
N = NA + NRES
E = 192000
NW = 32
K = 240
EB = 1920
NB = 1000
NPAD = 12032

_MESH = plsc.VectorSubcoreMesh(core_axis_name="c", subcore_axis_name="s")


def _silu(x):
    return x * jax.nn.sigmoid(x)


def _ln(h, g, b):
    mu = jnp.mean(h, axis=-1, keepdims=True)
    va = jnp.mean((h - mu) ** 2, axis=-1, keepdims=True)
    return (h - mu) / jnp.sqrt(va + 1e-5) * g + b


def _tc_encoder(xf, p, in_wh, in_wt, in_b, wm0, t, tw, tb, brows):
    nrows, din = xf.shape
    half = H // 2
    freq = jnp.exp(jnp.arange(half, dtype=jnp.float32)
                   * (-np.log(10000.0) / (half - 1)))[None, :]

    def body(x_ref, w1, b1, g1, bb1, w2, b2, g2, bb2, inw, inwt, inb, wm,
             t_ref, fr_ref, tw_ref, tb_ref, x0_ref, xw_ref):
        x = x_ref[...]
        h = _silu(_ln(x @ w1[...] + b1[...], g1[...], bb1[...]))
        h = _ln(h @ w2[...] + b2[...], g2[...], bb2[...])
        te_arg = t_ref[...] * fr_ref[...]
        te = (jnp.sum(jnp.sin(te_arg) * tw_ref[0:1, :])
              + jnp.sum(jnp.cos(te_arg) * tw_ref[1:2, :]) + tb_ref[0, 0])
        x0 = h @ inw[...] + te * inwt[...] + inb[...]
        x0_ref[...] = x0
        xw_ref[...] = x0 @ wm[...]

    full = lambda a, b: pl.BlockSpec((a, b), lambda i: (0, 0))
    return pl.pallas_call(
        body,
        grid=(nrows // brows,),
        in_specs=[
            pl.BlockSpec((brows, din), lambda i: (i, 0)),
            full(din, half), full(1, half), full(1, half), full(1, half),
            full(half, H), full(1, H), full(1, H), full(1, H),
            full(H, H), full(1, H), full(1, H), full(H, H),
            full(1, 1), full(1, half), full(2, half), full(1, 1),
        ],
        out_specs=[
            pl.BlockSpec((brows, H), lambda i: (i, 0)),
            pl.BlockSpec((brows, H), lambda i: (i, 0)),
        ],
        out_shape=[
            jax.ShapeDtypeStruct((nrows, H), jnp.float32),
            jax.ShapeDtypeStruct((nrows, H), jnp.float32),
        ],
    )(xf, p['w1'], p['b1'][None], p['g1'][None], p['bb1'][None],
      p['w2'], p['b2'][None], p['g2'][None], p['bb2'][None],
      in_wh, in_wt, in_b, wm0, t[:, None], freq, tw, tb)


def _sc_pos_diff(pos128, src, dst):
    @functools.partial(
        pl.kernel,
        mesh=_MESH,
        out_type=jax.ShapeDtypeStruct((E, 128), jnp.float32),
        scratch_types=[
            pltpu.VMEM((K,), jnp.int32),
            pltpu.VMEM((K,), jnp.int32),
            pltpu.VMEM((K, 128), jnp.float32),
            pltpu.VMEM((K, 128), jnp.float32),
            pltpu.SemaphoreType.DMA,
            pltpu.SemaphoreType.DMA,
        ],
    )
    def k(pos_hbm, src_hbm, dst_hbm, out_hbm, si_v, di_v, a_v, b_v, s1, s2):
        wid = lax.axis_index("s") * 2 + lax.axis_index("c")
        ept = E // NW
        nslab = ept // K

        def body(i, _):
            base = wid * ept + i * K
            pltpu.sync_copy(src_hbm.at[pl.ds(base, K)], si_v)
            pltpu.sync_copy(dst_hbm.at[pl.ds(base, K)], di_v)
            ca = pltpu.async_copy(pos_hbm.at[si_v], a_v, s1)
            cb = pltpu.async_copy(pos_hbm.at[di_v], b_v, s2)
            ca.wait()
            cb.wait()

            def row(r, _):
                b_v[r, pl.ds(0, 16)] = b_v[r, pl.ds(0, 16)] - a_v[r, pl.ds(0, 16)]
                return 0

            lax.fori_loop(0, K, row, 0, unroll=4)
            pltpu.sync_copy(b_v, out_hbm.at[pl.ds(base, K)])
            return 0

        lax.fori_loop(0, nslab, body, 0)

    return k(pos128, src, dst)


def _tc_edge_geom(diff16, etype, etype_emb):
    means = jnp.linspace(float(np.exp(-CUT)), 1.0, NRBF)[None, :]
    beta = float(((2.0 / NRBF) * (1.0 - np.exp(-CUT))) ** -2)

    def body(diff_ref, et_ref, emb_ref, means_ref, efb_ref, wts_ref):
        dif = diff_ref[...]
        dx, dy, dz = dif[:, 0:1], dif[:, 1:2], dif[:, 2:3]
        d = jnp.sqrt(dx * dx + dy * dy + dz * dz + 1e-12)
        ux, uy, uz = dx / d, dy / d, dz / d
        C = jnp.where(d < CUT, 0.5 * (jnp.cos(jnp.pi * d / CUT) + 1.0), 0.0)
        rbf = jnp.exp(-beta * (jnp.exp(-d) - means_ref[...]) ** 2)
        et = et_ref[...][:, 0]
        emb = emb_ref[...]
        emb_sel = (jnp.where((et == 0)[:, None], emb[0][None, :], 0.0)
                   + jnp.where((et == 1)[:, None], emb[1][None, :], 0.0)
                   + jnp.where((et == 2)[:, None], emb[2][None, :], 0.0))
        efb_ref[...] = rbf * C + emb_sel
        z = jnp.zeros_like(C)
        wts_ref[...] = jnp.concatenate([
            C, C * ux, C * uy, C * uz,
            C * ux * uy, C * uy * uz, C * uz * ux,
            C * (ux * ux - uy * uy), C * (3.0 * uz * uz - 1.0),
            z, z, z, z, z, z, z], axis=1)

    return pl.pallas_call(
        body,
        grid=(E // EB,),
        in_specs=[
            pl.BlockSpec((EB, 128), lambda i: (i, 0)),
            pl.BlockSpec((EB, 1), lambda i: (i, 0)),
            pl.BlockSpec((3, NRBF), lambda i: (0, 0)),
            pl.BlockSpec((1, NRBF), lambda i: (0, 0)),
        ],
        out_specs=[
            pl.BlockSpec((EB, NRBF), lambda i: (i, 0)),
            pl.BlockSpec((EB, 16), lambda i: (i, 0)),
        ],
        out_shape=[
            jax.ShapeDtypeStruct((E, NRBF), jnp.float32),
            jax.ShapeDtypeStruct((E, 16), jnp.float32),
        ],
    )(diff16, etype[:, None], etype_emb, means)


def _tc_edge_dense(efb, we, be, wm, bm):
    def body(efb_ref, we_r, be_r, wm_r, bm_r, out_ref):
        ef = _silu(efb_ref[...] @ we_r[...] + be_r[...])
        out_ref[...] = ef @ wm_r[...] + bm_r[...]

    return pl.pallas_call(
        body,
        grid=(E // EB,),
        in_specs=[
            pl.BlockSpec((EB, NRBF), lambda i: (i, 0)),
            pl.BlockSpec((NRBF, H), lambda i: (0, 0)),
            pl.BlockSpec((1, H), lambda i: (0, 0)),
            pl.BlockSpec((H, H), lambda i: (0, 0)),
            pl.BlockSpec((1, H), lambda i: (0, 0)),
        ],
        out_specs=pl.BlockSpec((EB, H), lambda i: (i, 0)),
        out_shape=jax.ShapeDtypeStruct((E, H), jnp.float32),
    )(efb, we, be[None], wm, bm[None])


def _sc_gather_silu(xw, efw, src):
    xw_pad = jnp.pad(xw, ((0, NPAD - xw.shape[0]), (0, 0)))
    KG = 120

    @functools.partial(
        pl.kernel,
        mesh=_MESH,
        out_type=jax.ShapeDtypeStruct((E, H), jnp.float32),
        scratch_types=[
            pltpu.VMEM((KG,), jnp.int32),
            pltpu.VMEM((KG, H), jnp.float32),
            pltpu.VMEM((KG, H), jnp.float32),
            pltpu.SemaphoreType.DMA,
            pltpu.VMEM_SHARED((NPAD, H), jnp.float32),
        ],
    )
    def k(xw_hbm, efw_hbm, src_hbm, out_hbm, idx_v, g_v, e_v, sem, xws):
        cid = lax.axis_index("c")
        sid = lax.axis_index("s")
        wid = sid * 2 + cid
        trows = NPAD // 16
        pltpu.sync_copy(xw_hbm.at[pl.ds(sid * trows, trows)],
                        xws.at[pl.ds(sid * trows, trows)])
        plsc.subcore_barrier()
        ept = E // NW
        nslab = ept // KG

        def body(i, _):
            base = wid * ept + i * KG
            pltpu.sync_copy(src_hbm.at[pl.ds(base, KG)], idx_v)
            ca = pltpu.async_copy(xws.at[idx_v], g_v, sem)
            pltpu.sync_copy(efw_hbm.at[pl.ds(base, KG)], e_v)
            ca.wait()

            def row(r, _):
                for j in range(H // 16):
                    p = g_v[r, pl.ds(j * 16, 16)] + e_v[r, pl.ds(j * 16, 16)]
                    e_v[r, pl.ds(j * 16, 16)] = p / (1.0 + jnp.exp(-p))
                return 0

            lax.fori_loop(0, KG, row, 0)
            pltpu.sync_copy(e_v, out_hbm.at[pl.ds(base, KG)])
            return 0

        lax.fori_loop(0, nslab, body, 0)

    return k(xw_pad, efw, src)


def _sc_scatter9(sil, wts_flat, dst):
    KS = 160

    @functools.partial(
        pl.kernel,
        mesh=_MESH,
        out_type=jax.ShapeDtypeStruct((9 * NPAD, H), jnp.float32),
        scratch_types=[
            pltpu.VMEM((KS,), jnp.int32),
            pltpu.VMEM((KS,), jnp.float32),
            pltpu.VMEM((KS, H), jnp.float32),
            pltpu.VMEM((16, H), jnp.float32),
            pltpu.VMEM_SHARED((NPAD, H), jnp.float32),
        ],
    )
    def k(sil_hbm, wts_hbm, dst_hbm, out_hbm, didx_v, w_v, m_v, z_v, acc):
        cid = lax.axis_index("c")
        sid = lax.axis_index("s")
        ept = E // 16
        nslab = ept // KS
        trows = NPAD // 16

        for r in range(16):
            for j in range(H // 16):
                z_v[r, pl.ds(j * 16, 16)] = jnp.zeros((16,), jnp.float32)

        def one_pass(p, _):
            ci = p * 2 + cid

            @pl.when(ci < 9)
            def _():
                def zbody(i, _):
                    pltpu.sync_copy(z_v, acc.at[pl.ds(sid * trows + i * 16, 16)])
                    return 0

                lax.fori_loop(0, trows // 16, zbody, 0)
                plsc.subcore_barrier()

                def body(i, _):
                    base = sid * ept + i * KS
                    pltpu.sync_copy(dst_hbm.at[pl.ds(base, KS)], didx_v)
                    pltpu.sync_copy(sil_hbm.at[pl.ds(base, KS)], m_v)
                    pltpu.sync_copy(wts_hbm.at[pl.ds(ci * E + base, KS)], w_v)

                    def grp(g, _):
                        w16 = w_v[pl.ds(g * 16, 16)]
                        for l in range(16):
                            spl = w16.at[jnp.full((16,), l, jnp.int32)].get(
                                mode='promise_in_bounds')
                            for j in range(H // 16):
                                m_v[g * 16 + l, pl.ds(j * 16, 16)] = (
                                    m_v[g * 16 + l, pl.ds(j * 16, 16)] * spl)
                        return 0

                    lax.fori_loop(0, KS // 16, grp, 0)
                    pltpu.sync_copy(m_v, acc.at[didx_v], add=True)
                    return 0

                lax.fori_loop(0, nslab, body, 0)
                plsc.subcore_barrier()
                pltpu.sync_copy(acc.at[pl.ds(sid * trows, trows)],
                                out_hbm.at[pl.ds(ci * NPAD + sid * trows, trows)])
                plsc.subcore_barrier()

            return 0

        lax.fori_loop(0, 5, one_pass, 0)

    return k(sil, wts_flat, dst)


def _tc_node_update(x, dv0, wu, bu, wm_next):
    def body(x_ref, dv_ref, wu_r, bu_r, wm_r, xo_ref, xw_ref):
        y = x_ref[...] + dv_ref[...]
        y = y + _silu(y @ wu_r[...] + bu_r[...])
        xo_ref[...] = y
        xw_ref[...] = y @ wm_r[...]

    return pl.pallas_call(
        body,
        grid=(N // NB,),
        in_specs=[
            pl.BlockSpec((NB, H), lambda i: (i, 0)),
            pl.BlockSpec((NB, H), lambda i: (i, 0)),
            pl.BlockSpec((H, H), lambda i: (0, 0)),
            pl.BlockSpec((1, H), lambda i: (0, 0)),
            pl.BlockSpec((H, H), lambda i: (0, 0)),
        ],
        out_specs=[
            pl.BlockSpec((NB, H), lambda i: (i, 0)),
            pl.BlockSpec((NB, H), lambda i: (i, 0)),
        ],
        out_shape=[
            jax.ShapeDtypeStruct((N, H), jnp.float32),
            jax.ShapeDtypeStruct((N, H), jnp.float32),
        ],
    )(x, dv0, wu, bu[None], wm_next)


def _geb_blk(x, v, wv1, wv2, u1w, u1b, u2w, u2b, dout, scalar_act):
    nv, nb, din = v.shape
    vec1 = (v.reshape(nv * nb, din) @ wv1).reshape(nv, nb, din)
    vec1n = jnp.sqrt(jnp.sum(vec1 * vec1, axis=0) + 1e-12)
    vec2 = (v.reshape(nv * nb, din) @ wv2).reshape(nv, nb, wv2.shape[1])
    h = jnp.concatenate([x, vec1n], axis=-1)
    h = _silu(h @ u1w + u1b)
    h = h @ u2w + u2b
    xo, gate = h[:, :dout], h[:, dout:]
    vo = vec2 * gate[None, :, :]
    if scalar_act:
        xo = _silu(xo)
    return xo, vo


def _tc_head(x, dvv, cw, fw):
    def body(x_ref, d1, d2, d3, d4,
             c_wv1, c_wv2, c_u1w, c_u1b, c_u2w, c_u2b,
             c2_wv1, c2_wv2, c2_u1w, c2_u1b, c2_u2w, c2_u2b,
             sn_w1, sn_b1, sn_w2, sn_b2, l1_w, l1_b,
             l2_w1, l2_b1, l2_w2, l2_b2, comb,
             f_wv1, f_wv2, f_u1w, f_u1b, f_u2w, f_u2b,
             f2_wv1, f2_wv2, f2_u1w, f2_u1b, f2_u2w, f2_u2b,
             vel_ref, xf_ref):
        x = x_ref[...]
        v = d1[...] + d2[...] + d3[...] + d4[...]
        fsel0 = lambda r: r[...][0]
        xc, vc = _geb_blk(x, v, c_wv1[...], c_wv2[...], c_u1w[...],
                          c_u1b[...], c_u2w[...], c_u2b[...], H // 2, True)
        xc, vc = _geb_blk(xc, vc, c2_wv1[...], c2_wv2[...], c2_u1w[...],
                          c2_u1b[...], c2_u2w[...], c2_u2b[...], H // 4, True)
        l1w = l1_w[...]
        l1v = jnp.concatenate(
            [jnp.sum(vc[kk] * l1w, axis=-1, keepdims=True) for kk in range(3)],
            axis=1) + l1_b[...]
        vl2 = vc[3:8]
        l2m = jnp.sqrt(jnp.sum(vl2 * vl2, axis=0) + 1e-12)
        l2mod = jnp.tanh(_silu(l2m @ l2_w1[...] + l2_b1[...]) @ l2_w2[...] + l2_b2[...])
        mag = jax.nn.sigmoid(_silu(xc @ sn_w1[...] + sn_b1[...]) @ sn_w2[...] + sn_b2[...])
        c0 = comb[0, 0]
        c1 = comb[0, 1]
        e0 = jnp.exp(c0 - jnp.maximum(c0, c1))
        e1 = jnp.exp(c1 - jnp.maximum(c0, c1))
        w0 = e0 / (e0 + e1)
        w1 = e1 / (e0 + e1)
        vel = mag * ((w0 + w1 * l2mod) * l1v)
        vel_ref[...] = jnp.concatenate([vel, jnp.zeros((vel.shape[0], 5), jnp.float32)], axis=1)
        xa, va = _geb_blk(x, v, fsel0(f_wv1), fsel0(f_wv2), fsel0(f_u1w),
                          fsel0(f_u1b), fsel0(f_u2w), fsel0(f_u2b), H // 2, True)
        xa, _ = _geb_blk(xa, va, fsel0(f2_wv1), fsel0(f2_wv2), fsel0(f2_u1w),
                         fsel0(f2_u1b), fsel0(f2_u2w), fsel0(f2_u2b), RNF, False)
        xf_ref[...] = xa

    full = lambda *s: pl.BlockSpec(s, lambda i: (0,) * len(s))
    fsel = lambda *s: pl.BlockSpec((1,) + s[1:],
                                   lambda i: (i // 10,) + (0,) * (len(s) - 1))
    dspec = pl.BlockSpec((NV, NB, H), lambda i: (0, i, 0))
    vel, xf = pl.pallas_call(
        body,
        grid=(N // NB,),
        in_specs=[
            pl.BlockSpec((NB, H), lambda i: (i, 0)),
            dspec, dspec, dspec, dspec,
            full(H, H), full(H, H // 2), full(2 * H, H), full(1, H),
            full(H, H), full(1, H),
            full(H // 2, H // 2), full(H // 2, H // 4), full(H, H // 2),
            full(1, H // 2), full(H // 2, H // 2), full(1, H // 2),
            full(H // 4, H // 8), full(1, H // 8), full(H // 8, 1), full(1, 1),
            full(1, H // 4), full(1, 1),
            full(H // 4, H // 8), full(1, H // 8), full(H // 8, 1), full(1, 1),
            full(1, 2),
            fsel(2, H, H), fsel(2, H, H // 2), fsel(2, 2 * H, H), fsel(2, 1, H),
            fsel(2, H, H), fsel(2, 1, H),
            fsel(2, H // 2, H // 2), fsel(2, H // 2, RNF), fsel(2, H, H // 2),
            fsel(2, 1, H // 2), fsel(2, H // 2, 2 * RNF), fsel(2, 1, 2 * RNF),
        ],
        out_specs=[
            pl.BlockSpec((NB, 8), lambda i: (i, 0)),
            pl.BlockSpec((NB, RNF), lambda i: (i, 0)),
        ],
        out_shape=[
            jax.ShapeDtypeStruct((N, 8), jnp.float32),
            jax.ShapeDtypeStruct((N, RNF), jnp.float32),
        ],
    )(x, dvv[0], dvv[1], dvv[2], dvv[3], *cw, *fw)
    return vel, xf


def _pad_cols(a, w):
    return jnp.pad(a, ((0, 0), (0, w - a.shape[1])))


def kernel(xh_atoms, xh_residues, t, mask_atoms, mask_residues, edge_index, edge_types, params):
    src = edge_index[0].astype(jnp.int32)
    dst = edge_index[1].astype(jnp.int32)
    etype = edge_types.astype(jnp.int32)
    lp = params['layers']
    in_w = params['in_w']
    tw = jnp.stack([params['time_w'][:H // 2, 0], params['time_w'][H // 2:, 0]])
    tb = params['time_b'][None]

    x0a, xwa = _tc_encoder(xh_atoms[:, 3:], params['atom_enc'], in_w[:H],
                           in_w[H:H + 1], params['in_b'][None], lp[0]['wm'],
                           t, tw, tb, 2000)
    x0r, xwr = _tc_encoder(xh_residues[:, 3:], params['res_enc'], in_w[:H],
                           in_w[H:H + 1], params['in_b'][None], lp[0]['wm'],
                           t, tw, tb, 2000)
    x = jnp.concatenate([x0a, x0r], axis=0)
    xw = jnp.concatenate([xwa, xwr], axis=0)

    pos128 = jnp.zeros((N, 128), jnp.float32).at[:, :3].set(
        jnp.concatenate([xh_atoms[:, :3], xh_residues[:, :3]], axis=0))
    diff = _sc_pos_diff(pos128, src, dst)
    efb, wts_e = _tc_edge_geom(diff, etype, params['etype_emb'])
    wts_flat = wts_e.T.reshape(-1)

    dvvs = []
    for li in range(NL):
        p = lp[li]
        efw = _tc_edge_dense(efb, p['we'], p['be'], p['wm'], p['bm'])
        sil = _sc_gather_silu(xw, efw, src)
        dv9 = _sc_scatter9(sil, wts_flat, dst).reshape(9, NPAD, H)
        dvvs.append(dv9[1:9, :N])
        wm_next = lp[li + 1]['wm'] if li + 1 < NL else p['wu']
        x, xw = _tc_node_update(x, dv9[0, :N], p['wu'], p['bu'], wm_next)

    cp = params['coord']
    cw = []
    for g, dout in ((cp['g1'], H // 2), (cp['g2'], H // 4)):
        cw += [g['wv1'], g['wv2'], g['u1w'], g['u1b'][None], g['u2w'], g['u2b'][None]]
    cw += [cp['sn_w1'], cp['sn_b1'][None], cp['sn_w2'], cp['sn_b2'][None],
           cp['l1_w'].T, cp['l1_b'][None],
           cp['l2_w1'], cp['l2_b1'][None], cp['l2_w2'], cp['l2_b2'][None],
           cp['comb'][None]]

    fa, fr = params['feat_a'], params['feat_r']
    fw = []
    for key in ('wv1', 'wv2', 'u1w', 'u1b', 'u2w', 'u2b'):
        a, r = fa['g1'][key], fr['g1'][key]
        if a.ndim == 1:
            a, r = a[None], r[None]
        fw.append(jnp.stack([a, r]))
    a2, r2 = fa['g2'], fr['g2']
    u2w_a = jnp.concatenate([_pad_cols(a2['u2w'][:, :ANF], RNF),
                             _pad_cols(a2['u2w'][:, ANF:], RNF)], axis=1)
    u2b_a = jnp.concatenate([jnp.pad(a2['u2b'][:ANF], (0, RNF - ANF)),
                             jnp.pad(a2['u2b'][ANF:], (0, RNF - ANF))])
    for key, aw, rw in (('wv1', a2['wv1'], r2['wv1']),
                        ('wv2', _pad_cols(a2['wv2'], RNF), r2['wv2']),
                        ('u1w', a2['u1w'], r2['u1w']),
                        ('u1b', a2['u1b'][None], r2['u1b'][None]),
                        ('u2w', u2w_a, r2['u2w']),
                        ('u2b', u2b_a[None], r2['u2b'][None])):
        fw.append(jnp.stack([aw, rw]))

    vel, xf = _tc_head(x, dvvs, cw, fw)
    out_a = jnp.concatenate([vel[:NA, :3], xf[:NA, :ANF]], axis=-1)
    out_r = jnp.concatenate([vel[NA:, :3], xf[NA:, :RNF]], axis=-1)
    return (out_a, out_r)


# speedup vs baseline: 3.1829x; 1.7020x over previous
import functools

import jax
import jax.numpy as jnp
import numpy as np
from jax import lax
from jax.experimental import pallas as pl
from jax.experimental.pallas import tpu as pltpu
from jax.experimental.pallas import tpu_sc as plsc

H = 128
NRBF = 32
NL = 4
ANF = 16
RNF = 21
NV = 8
CUT = 8.0
NA = 10000
NRES = 2000
N = NA + NRES
E = 192000
NW = 32
K = 240
EB = 1920
NB = 1000
NPAD = 12032

_MESH = plsc.VectorSubcoreMesh(core_axis_name="c", subcore_axis_name="s")


def _silu(x):
    return x * jax.nn.sigmoid(x)


def _ln(h, g, b):
    mu = jnp.mean(h, axis=-1, keepdims=True)
    va = jnp.mean((h - mu) ** 2, axis=-1, keepdims=True)
    return (h - mu) / jnp.sqrt(va + 1e-5) * g + b


def _tc_encoder(xf, p, in_wh, in_wt, in_b, wm0, t, tw, tb, brows):
    nrows, din = xf.shape
    half = H // 2
    freq = jnp.exp(jnp.arange(half, dtype=jnp.float32)
                   * (-np.log(10000.0) / (half - 1)))[None, :]

    def body(x_ref, w1, b1, g1, bb1, w2, b2, g2, bb2, inw, inwt, inb, wm,
             t_ref, fr_ref, tw_ref, tb_ref, x0_ref, xw_ref):
        x = x_ref[...]
        h = _silu(_ln(x @ w1[...] + b1[...], g1[...], bb1[...]))
        h = _ln(h @ w2[...] + b2[...], g2[...], bb2[...])
        te_arg = t_ref[...] * fr_ref[...]
        te = (jnp.sum(jnp.sin(te_arg) * tw_ref[0:1, :])
              + jnp.sum(jnp.cos(te_arg) * tw_ref[1:2, :]) + tb_ref[0, 0])
        x0 = h @ inw[...] + te * inwt[...] + inb[...]
        x0_ref[...] = x0
        xw_ref[...] = x0 @ wm[...]

    full = lambda a, b: pl.BlockSpec((a, b), lambda i: (0, 0))
    return pl.pallas_call(
        body,
        grid=(nrows // brows,),
        in_specs=[
            pl.BlockSpec((brows, din), lambda i: (i, 0)),
            full(din, half), full(1, half), full(1, half), full(1, half),
            full(half, H), full(1, H), full(1, H), full(1, H),
            full(H, H), full(1, H), full(1, H), full(H, H),
            full(1, 1), full(1, half), full(2, half), full(1, 1),
        ],
        out_specs=[
            pl.BlockSpec((brows, H), lambda i: (i, 0)),
            pl.BlockSpec((brows, H), lambda i: (i, 0)),
        ],
        out_shape=[
            jax.ShapeDtypeStruct((nrows, H), jnp.float32),
            jax.ShapeDtypeStruct((nrows, H), jnp.float32),
        ],
    )(xf, p['w1'], p['b1'][None], p['g1'][None], p['bb1'][None],
      p['w2'], p['b2'][None], p['g2'][None], p['bb2'][None],
      in_wh, in_wt, in_b, wm0, t[:, None], freq, tw, tb)


def _sc_pos_diff(pos128, src, dst):
    @functools.partial(
        pl.kernel,
        mesh=_MESH,
        out_type=jax.ShapeDtypeStruct((E, 128), jnp.float32),
        scratch_types=[
            pltpu.VMEM((K,), jnp.int32),
            pltpu.VMEM((K,), jnp.int32),
            pltpu.VMEM((K, 128), jnp.float32),
            pltpu.VMEM((K, 128), jnp.float32),
            pltpu.SemaphoreType.DMA,
            pltpu.SemaphoreType.DMA,
        ],
    )
    def k(pos_hbm, src_hbm, dst_hbm, out_hbm, si_v, di_v, a_v, b_v, s1, s2):
        wid = lax.axis_index("s") * 2 + lax.axis_index("c")
        ept = E // NW
        nslab = ept // K

        def body(i, _):
            base = wid * ept + i * K
            pltpu.sync_copy(src_hbm.at[pl.ds(base, K)], si_v)
            pltpu.sync_copy(dst_hbm.at[pl.ds(base, K)], di_v)
            ca = pltpu.async_copy(pos_hbm.at[si_v], a_v, s1)
            cb = pltpu.async_copy(pos_hbm.at[di_v], b_v, s2)
            ca.wait()
            cb.wait()

            def row(r, _):
                b_v[r, pl.ds(0, 16)] = b_v[r, pl.ds(0, 16)] - a_v[r, pl.ds(0, 16)]
                return 0

            lax.fori_loop(0, K, row, 0, unroll=4)
            pltpu.sync_copy(b_v, out_hbm.at[pl.ds(base, K)])
            return 0

        lax.fori_loop(0, nslab, body, 0)

    return k(pos128, src, dst)


def _tc_edge_geom(diff16, etype, etype_emb):
    means = jnp.linspace(float(np.exp(-CUT)), 1.0, NRBF)[None, :]
    beta = float(((2.0 / NRBF) * (1.0 - np.exp(-CUT))) ** -2)

    def body(diff_ref, et_ref, emb_ref, means_ref, efb_ref, wts_ref):
        dif = diff_ref[...]
        dx, dy, dz = dif[:, 0:1], dif[:, 1:2], dif[:, 2:3]
        d = jnp.sqrt(dx * dx + dy * dy + dz * dz + 1e-12)
        ux, uy, uz = dx / d, dy / d, dz / d
        C = jnp.where(d < CUT, 0.5 * (jnp.cos(jnp.pi * d / CUT) + 1.0), 0.0)
        rbf = jnp.exp(-beta * (jnp.exp(-d) - means_ref[...]) ** 2)
        et = et_ref[...][:, 0]
        emb = emb_ref[...]
        emb_sel = (jnp.where((et == 0)[:, None], emb[0][None, :], 0.0)
                   + jnp.where((et == 1)[:, None], emb[1][None, :], 0.0)
                   + jnp.where((et == 2)[:, None], emb[2][None, :], 0.0))
        efb_ref[...] = rbf * C + emb_sel
        z = jnp.zeros_like(C)
        wts_ref[...] = jnp.concatenate([
            C, C * ux, C * uy, C * uz,
            C * ux * uy, C * uy * uz, C * uz * ux,
            C * (ux * ux - uy * uy), C * (3.0 * uz * uz - 1.0),
            z, z, z, z, z, z, z], axis=1)

    return pl.pallas_call(
        body,
        grid=(E // EB,),
        in_specs=[
            pl.BlockSpec((EB, 128), lambda i: (i, 0)),
            pl.BlockSpec((EB, 1), lambda i: (i, 0)),
            pl.BlockSpec((3, NRBF), lambda i: (0, 0)),
            pl.BlockSpec((1, NRBF), lambda i: (0, 0)),
        ],
        out_specs=[
            pl.BlockSpec((EB, NRBF), lambda i: (i, 0)),
            pl.BlockSpec((EB, 16), lambda i: (i, 0)),
        ],
        out_shape=[
            jax.ShapeDtypeStruct((E, NRBF), jnp.float32),
            jax.ShapeDtypeStruct((E, 16), jnp.float32),
        ],
    )(diff16, etype[:, None], etype_emb, means)


def _tc_edge_dense(efb, we, be, wm, bm):
    def body(efb_ref, we_r, be_r, wm_r, bm_r, out_ref):
        ef = _silu(efb_ref[...] @ we_r[...] + be_r[...])
        out_ref[...] = ef @ wm_r[...] + bm_r[...]

    return pl.pallas_call(
        body,
        grid=(E // EB,),
        in_specs=[
            pl.BlockSpec((EB, NRBF), lambda i: (i, 0)),
            pl.BlockSpec((NRBF, H), lambda i: (0, 0)),
            pl.BlockSpec((1, H), lambda i: (0, 0)),
            pl.BlockSpec((H, H), lambda i: (0, 0)),
            pl.BlockSpec((1, H), lambda i: (0, 0)),
        ],
        out_specs=pl.BlockSpec((EB, H), lambda i: (i, 0)),
        out_shape=jax.ShapeDtypeStruct((E, H), jnp.float32),
    )(efb, we, be[None], wm, bm[None])


def _sc_gather_silu(xw, efw, src):
    xw_pad = jnp.pad(xw, ((0, NPAD - xw.shape[0]), (0, 0)))
    KG = 120

    @functools.partial(
        pl.kernel,
        mesh=_MESH,
        out_type=jax.ShapeDtypeStruct((E, H), jnp.float32),
        scratch_types=[
            pltpu.VMEM((KG,), jnp.int32),
            pltpu.VMEM((KG, H), jnp.float32),
            pltpu.VMEM((KG, H), jnp.float32),
            pltpu.SemaphoreType.DMA,
            pltpu.VMEM_SHARED((NPAD, H), jnp.float32),
        ],
    )
    def k(xw_hbm, efw_hbm, src_hbm, out_hbm, idx_v, g_v, e_v, sem, xws):
        cid = lax.axis_index("c")
        sid = lax.axis_index("s")
        wid = sid * 2 + cid
        trows = NPAD // 16
        pltpu.sync_copy(xw_hbm.at[pl.ds(sid * trows, trows)],
                        xws.at[pl.ds(sid * trows, trows)])
        plsc.subcore_barrier()
        ept = E // NW
        nslab = ept // KG

        def body(i, _):
            base = wid * ept + i * KG
            pltpu.sync_copy(src_hbm.at[pl.ds(base, KG)], idx_v)
            ca = pltpu.async_copy(xws.at[idx_v], g_v, sem)
            pltpu.sync_copy(efw_hbm.at[pl.ds(base, KG)], e_v)
            ca.wait()

            def row(r, _):
                for j in range(H // 16):
                    p = g_v[r, pl.ds(j * 16, 16)] + e_v[r, pl.ds(j * 16, 16)]
                    e_v[r, pl.ds(j * 16, 16)] = p / (1.0 + jnp.exp(-p))
                return 0

            lax.fori_loop(0, KG, row, 0)
            pltpu.sync_copy(e_v, out_hbm.at[pl.ds(base, KG)])
            return 0

        lax.fori_loop(0, nslab, body, 0)

    return k(xw_pad, efw, src)


def _sc_scatter0(sil, wts_flat, dst, s_in):
    first = s_in is None
    KS = 80
    scr = [
        pltpu.VMEM((KS,), jnp.int32),
        pltpu.VMEM((KS,), jnp.float32),
        pltpu.VMEM((KS, H), jnp.float32),
        pltpu.VMEM((KS, H), jnp.float32),
        pltpu.VMEM((16, H), jnp.float32),
        pltpu.VMEM_SHARED((NPAD, H), jnp.float32),
    ]

    @functools.partial(
        pl.kernel,
        mesh=_MESH,
        out_type=[
            jax.ShapeDtypeStruct((2 * NPAD, H), jnp.float32),
            jax.ShapeDtypeStruct((E, H), jnp.float32),
        ],
        scratch_types=scr,
    )
    def k(sil_hbm, wts_hbm, dst_hbm, *rest):
        if first:
            out_hbm, s_out = rest[:2]
            s_hbm = None
            didx_v, w_v, m_v, s_v, z_v, acc = rest[2:]
        else:
            s_hbm = rest[0]
            out_hbm, s_out = rest[1:3]
            didx_v, w_v, m_v, s_v, z_v, acc = rest[3:]
        cid = lax.axis_index("c")
        sid = lax.axis_index("s")
        ept = E // NW
        nslab = ept // KS
        trows = NPAD // 16

        for r in range(16):
            for j in range(H // 16):
                z_v[r, pl.ds(j * 16, 16)] = jnp.zeros((16,), jnp.float32)

        def zbody(i, _):
            pltpu.sync_copy(z_v, acc.at[pl.ds(sid * trows + i * 16, 16)])
            return 0

        lax.fori_loop(0, trows // 16, zbody, 0)
        plsc.subcore_barrier()

        def body(i, _):
            base = cid * (E // 2) + sid * ept + i * KS
            pltpu.sync_copy(dst_hbm.at[pl.ds(base, KS)], didx_v)
            pltpu.sync_copy(sil_hbm.at[pl.ds(base, KS)], m_v)
            pltpu.sync_copy(wts_hbm.at[pl.ds(base, KS)], w_v)
            if first:
                pltpu.sync_copy(m_v, s_out.at[pl.ds(base, KS)])
            else:
                pltpu.sync_copy(s_hbm.at[pl.ds(base, KS)], s_v)

            def grp(g, _):
                w16 = w_v[pl.ds(g * 16, 16)]
                for l in range(16):
                    spl = w16.at[jnp.full((16,), l, jnp.int32)].get(
                        mode='promise_in_bounds')
                    for j in range(H // 16):
                        rr = g * 16 + l
                        m = m_v[rr, pl.ds(j * 16, 16)]
                        if not first:
                            s_v[rr, pl.ds(j * 16, 16)] = (
                                s_v[rr, pl.ds(j * 16, 16)] + m)
                        m_v[rr, pl.ds(j * 16, 16)] = m * spl
                return 0

            lax.fori_loop(0, KS // 16, grp, 0)
            if not first:
                pltpu.sync_copy(s_v, s_out.at[pl.ds(base, KS)])
            pltpu.sync_copy(m_v, acc.at[didx_v], add=True)
            return 0

        lax.fori_loop(0, nslab, body, 0)
        plsc.subcore_barrier()
        pltpu.sync_copy(acc.at[pl.ds(sid * trows, trows)],
                        out_hbm.at[pl.ds(cid * NPAD + sid * trows, trows)])

    if first:
        return k(sil, wts_flat, dst)
    return k(sil, wts_flat, dst, s_in)


def _sc_scatter8(s, wts_flat, dst):
    KS = 160

    @functools.partial(
        pl.kernel,
        mesh=_MESH,
        out_type=jax.ShapeDtypeStruct((8 * NPAD, H), jnp.float32),
        scratch_types=[
            pltpu.VMEM((KS,), jnp.int32),
            pltpu.VMEM((KS,), jnp.float32),
            pltpu.VMEM((KS, H), jnp.float32),
            pltpu.VMEM((16, H), jnp.float32),
            pltpu.VMEM_SHARED((NPAD, H), jnp.float32),
        ],
    )
    def k(sil_hbm, wts_hbm, dst_hbm, out_hbm, didx_v, w_v, m_v, z_v, acc):
        cid = lax.axis_index("c")
        sid = lax.axis_index("s")
        ept = E // 16
        nslab = ept // KS
        trows = NPAD // 16

        for r in range(16):
            for j in range(H // 16):
                z_v[r, pl.ds(j * 16, 16)] = jnp.zeros((16,), jnp.float32)

        def one_pass(p, _):
            ci = p * 2 + cid

            def zbody(i, _):
                pltpu.sync_copy(z_v, acc.at[pl.ds(sid * trows + i * 16, 16)])
                return 0

            lax.fori_loop(0, trows // 16, zbody, 0)
            plsc.subcore_barrier()

            def body(i, _):
                base = sid * ept + i * KS
                pltpu.sync_copy(dst_hbm.at[pl.ds(base, KS)], didx_v)
                pltpu.sync_copy(sil_hbm.at[pl.ds(base, KS)], m_v)
                pltpu.sync_copy(wts_hbm.at[pl.ds((ci + 1) * E + base, KS)], w_v)

                def grp(g, _):
                    w16 = w_v[pl.ds(g * 16, 16)]
                    for l in range(16):
                        spl = w16.at[jnp.full((16,), l, jnp.int32)].get(
                            mode='promise_in_bounds')
                        for j in range(H // 16):
                            m_v[g * 16 + l, pl.ds(j * 16, 16)] = (
                                m_v[g * 16 + l, pl.ds(j * 16, 16)] * spl)
                    return 0

                lax.fori_loop(0, KS // 16, grp, 0)
                pltpu.sync_copy(m_v, acc.at[didx_v], add=True)
                return 0

            lax.fori_loop(0, nslab, body, 0)
            plsc.subcore_barrier()
            pltpu.sync_copy(acc.at[pl.ds(sid * trows, trows)],
                            out_hbm.at[pl.ds(ci * NPAD + sid * trows, trows)])
            plsc.subcore_barrier()
            return 0

        lax.fori_loop(0, 4, one_pass, 0)

    return k(s, wts_flat, dst)


def _tc_node_update(x, dv0a, dv0b, wu, bu, wm_next):
    def body(x_ref, dva_ref, dvb_ref, wu_r, bu_r, wm_r, xo_ref, xw_ref):
        y = x_ref[...] + dva_ref[...] + dvb_ref[...]
        y = y + _silu(y @ wu_r[...] + bu_r[...])
        xo_ref[...] = y
        xw_ref[...] = y @ wm_r[...]

    return pl.pallas_call(
        body,
        grid=(N // NB,),
        in_specs=[
            pl.BlockSpec((NB, H), lambda i: (i, 0)),
            pl.BlockSpec((NB, H), lambda i: (i, 0)),
            pl.BlockSpec((NB, H), lambda i: (i, 0)),
            pl.BlockSpec((H, H), lambda i: (0, 0)),
            pl.BlockSpec((1, H), lambda i: (0, 0)),
            pl.BlockSpec((H, H), lambda i: (0, 0)),
        ],
        out_specs=[
            pl.BlockSpec((NB, H), lambda i: (i, 0)),
            pl.BlockSpec((NB, H), lambda i: (i, 0)),
        ],
        out_shape=[
            jax.ShapeDtypeStruct((N, H), jnp.float32),
            jax.ShapeDtypeStruct((N, H), jnp.float32),
        ],
    )(x, dv0a, dv0b, wu, bu[None], wm_next)


def _geb_blk(x, v, wv1, wv2, u1w, u1b, u2w, u2b, dout, scalar_act):
    nv, nb, din = v.shape
    vec1 = (v.reshape(nv * nb, din) @ wv1).reshape(nv, nb, din)
    vec1n = jnp.sqrt(jnp.sum(vec1 * vec1, axis=0) + 1e-12)
    vec2 = (v.reshape(nv * nb, din) @ wv2).reshape(nv, nb, wv2.shape[1])
    h = jnp.concatenate([x, vec1n], axis=-1)
    h = _silu(h @ u1w + u1b)
    h = h @ u2w + u2b
    xo, gate = h[:, :dout], h[:, dout:]
    vo = vec2 * gate[None, :, :]
    if scalar_act:
        xo = _silu(xo)
    return xo, vo


def _tc_head(x, dv8, cw, fw):
    def body(x_ref, d1,
             c_wv1, c_wv2, c_u1w, c_u1b, c_u2w, c_u2b,
             c2_wv1, c2_wv2, c2_u1w, c2_u1b, c2_u2w, c2_u2b,
             sn_w1, sn_b1, sn_w2, sn_b2, l1_w, l1_b,
             l2_w1, l2_b1, l2_w2, l2_b2, comb,
             f_wv1, f_wv2, f_u1w, f_u1b, f_u2w, f_u2b,
             f2_wv1, f2_wv2, f2_u1w, f2_u1b, f2_u2w, f2_u2b,
             vel_ref, xf_ref):
        x = x_ref[...]
        v = d1[...]
        fsel0 = lambda r: r[...][0]
        xc, vc = _geb_blk(x, v, c_wv1[...], c_wv2[...], c_u1w[...],
                          c_u1b[...], c_u2w[...], c_u2b[...], H // 2, True)
        xc, vc = _geb_blk(xc, vc, c2_wv1[...], c2_wv2[...], c2_u1w[...],
                          c2_u1b[...], c2_u2w[...], c2_u2b[...], H // 4, True)
        l1w = l1_w[...]
        l1v = jnp.concatenate(
            [jnp.sum(vc[kk] * l1w, axis=-1, keepdims=True) for kk in range(3)],
            axis=1) + l1_b[...]
        vl2 = vc[3:8]
        l2m = jnp.sqrt(jnp.sum(vl2 * vl2, axis=0) + 1e-12)
        l2mod = jnp.tanh(_silu(l2m @ l2_w1[...] + l2_b1[...]) @ l2_w2[...] + l2_b2[...])
        mag = jax.nn.sigmoid(_silu(xc @ sn_w1[...] + sn_b1[...]) @ sn_w2[...] + sn_b2[...])
        c0 = comb[0, 0]
        c1 = comb[0, 1]
        e0 = jnp.exp(c0 - jnp.maximum(c0, c1))
        e1 = jnp.exp(c1 - jnp.maximum(c0, c1))
        w0 = e0 / (e0 + e1)
        w1 = e1 / (e0 + e1)
        vel = mag * ((w0 + w1 * l2mod) * l1v)
        vel_ref[...] = jnp.concatenate([vel, jnp.zeros((vel.shape[0], 5), jnp.float32)], axis=1)
        xa, va = _geb_blk(x, v, fsel0(f_wv1), fsel0(f_wv2), fsel0(f_u1w),
                          fsel0(f_u1b), fsel0(f_u2w), fsel0(f_u2b), H // 2, True)
        xa, _ = _geb_blk(xa, va, fsel0(f2_wv1), fsel0(f2_wv2), fsel0(f2_u1w),
                         fsel0(f2_u1b), fsel0(f2_u2w), fsel0(f2_u2b), RNF, False)
        xf_ref[...] = xa

    full = lambda *s: pl.BlockSpec(s, lambda i: (0,) * len(s))
    fsel = lambda *s: pl.BlockSpec((1,) + s[1:],
                                   lambda i: (i // 10,) + (0,) * (len(s) - 1))
    dspec = pl.BlockSpec((NV, NB, H), lambda i: (0, i, 0))
    vel, xf = pl.pallas_call(
        body,
        grid=(N // NB,),
        in_specs=[
            pl.BlockSpec((NB, H), lambda i: (i, 0)),
            dspec,
            full(H, H), full(H, H // 2), full(2 * H, H), full(1, H),
            full(H, H), full(1, H),
            full(H // 2, H // 2), full(H // 2, H // 4), full(H, H // 2),
            full(1, H // 2), full(H // 2, H // 2), full(1, H // 2),
            full(H // 4, H // 8), full(1, H // 8), full(H // 8, 1), full(1, 1),
            full(1, H // 4), full(1, 1),
            full(H // 4, H // 8), full(1, H // 8), full(H // 8, 1), full(1, 1),
            full(1, 2),
            fsel(2, H, H), fsel(2, H, H // 2), fsel(2, 2 * H, H), fsel(2, 1, H),
            fsel(2, H, H), fsel(2, 1, H),
            fsel(2, H // 2, H // 2), fsel(2, H // 2, RNF), fsel(2, H, H // 2),
            fsel(2, 1, H // 2), fsel(2, H // 2, 2 * RNF), fsel(2, 1, 2 * RNF),
        ],
        out_specs=[
            pl.BlockSpec((NB, 8), lambda i: (i, 0)),
            pl.BlockSpec((NB, RNF), lambda i: (i, 0)),
        ],
        out_shape=[
            jax.ShapeDtypeStruct((N, 8), jnp.float32),
            jax.ShapeDtypeStruct((N, RNF), jnp.float32),
        ],
    )(x, dv8, *cw, *fw)
    return vel, xf


def _pad_cols(a, w):
    return jnp.pad(a, ((0, 0), (0, w - a.shape[1])))


def kernel(xh_atoms, xh_residues, t, mask_atoms, mask_residues, edge_index, edge_types, params):
    src = edge_index[0].astype(jnp.int32)
    dst = edge_index[1].astype(jnp.int32)
    etype = edge_types.astype(jnp.int32)
    lp = params['layers']
    in_w = params['in_w']
    tw = jnp.stack([params['time_w'][:H // 2, 0], params['time_w'][H // 2:, 0]])
    tb = params['time_b'][None]

    x0a, xwa = _tc_encoder(xh_atoms[:, 3:], params['atom_enc'], in_w[:H],
                           in_w[H:H + 1], params['in_b'][None], lp[0]['wm'],
                           t, tw, tb, 2000)
    x0r, xwr = _tc_encoder(xh_residues[:, 3:], params['res_enc'], in_w[:H],
                           in_w[H:H + 1], params['in_b'][None], lp[0]['wm'],
                           t, tw, tb, 2000)
    x = jnp.concatenate([x0a, x0r], axis=0)
    xw = jnp.concatenate([xwa, xwr], axis=0)

    pos128 = jnp.zeros((N, 128), jnp.float32).at[:, :3].set(
        jnp.concatenate([xh_atoms[:, :3], xh_residues[:, :3]], axis=0))
    diff = _sc_pos_diff(pos128, src, dst)
    efb, wts_e = _tc_edge_geom(diff, etype, params['etype_emb'])
    wts_flat = wts_e.T.reshape(-1)

    s = None
    for li in range(NL):
        p = lp[li]
        efw = _tc_edge_dense(efb, p['we'], p['be'], p['wm'], p['bm'])
        sil = _sc_gather_silu(xw, efw, src)
        dv0p, s = _sc_scatter0(sil, wts_flat, dst, s)
        dv0p = dv0p.reshape(2, NPAD, H)
        wm_next = lp[li + 1]['wm'] if li + 1 < NL else p['wu']
        x, xw = _tc_node_update(x, dv0p[0, :N], dv0p[1, :N],
                                p['wu'], p['bu'], wm_next)
    dv8 = _sc_scatter8(s, wts_flat, dst).reshape(8, NPAD, H)[:, :N]

    cp = params['coord']
    cw = []
    for g, dout in ((cp['g1'], H // 2), (cp['g2'], H // 4)):
        cw += [g['wv1'], g['wv2'], g['u1w'], g['u1b'][None], g['u2w'], g['u2b'][None]]
    cw += [cp['sn_w1'], cp['sn_b1'][None], cp['sn_w2'], cp['sn_b2'][None],
           cp['l1_w'].T, cp['l1_b'][None],
           cp['l2_w1'], cp['l2_b1'][None], cp['l2_w2'], cp['l2_b2'][None],
           cp['comb'][None]]

    fa, fr = params['feat_a'], params['feat_r']
    fw = []
    for key in ('wv1', 'wv2', 'u1w', 'u1b', 'u2w', 'u2b'):
        a, r = fa['g1'][key], fr['g1'][key]
        if a.ndim == 1:
            a, r = a[None], r[None]
        fw.append(jnp.stack([a, r]))
    a2, r2 = fa['g2'], fr['g2']
    u2w_a = jnp.concatenate([_pad_cols(a2['u2w'][:, :ANF], RNF),
                             _pad_cols(a2['u2w'][:, ANF:], RNF)], axis=1)
    u2b_a = jnp.concatenate([jnp.pad(a2['u2b'][:ANF], (0, RNF - ANF)),
                             jnp.pad(a2['u2b'][ANF:], (0, RNF - ANF))])
    for key, aw, rw in (('wv1', a2['wv1'], r2['wv1']),
                        ('wv2', _pad_cols(a2['wv2'], RNF), r2['wv2']),
                        ('u1w', a2['u1w'], r2['u1w']),
                        ('u1b', a2['u1b'][None], r2['u1b'][None]),
                        ('u2w', u2w_a, r2['u2w']),
                        ('u2b', u2b_a[None], r2['u2b'][None])):
        fw.append(jnp.stack([aw, rw]))

    vel, xf = _tc_head(x, dv8, cw, fw)
    out_a = jnp.concatenate([vel[:NA, :3], xf[:NA, :ANF]], axis=-1)
    out_r = jnp.concatenate([vel[NA:, :3], xf[NA:, :RNF]], axis=-1)
    return (out_a, out_r)


# speedup vs baseline: 3.4058x; 1.0700x over previous
import functools

import jax
import jax.numpy as jnp
import numpy as np
from jax import lax
from jax.experimental import pallas as pl
from jax.experimental.pallas import tpu as pltpu
from jax.experimental.pallas import tpu_sc as plsc

H = 128
NRBF = 32
NL = 4
ANF = 16
RNF = 21
NV = 8
CUT = 8.0
NA = 10000
NRES = 2000
N = NA + NRES
E = 192000
NW = 32
K = 240
EB = 1920
NB = 1000
NPAD = 12032

_MESH = plsc.VectorSubcoreMesh(core_axis_name="c", subcore_axis_name="s")


def _silu(x):
    return x * jax.nn.sigmoid(x)


def _ln(h, g, b):
    mu = jnp.mean(h, axis=-1, keepdims=True)
    va = jnp.mean((h - mu) ** 2, axis=-1, keepdims=True)
    return (h - mu) / jnp.sqrt(va + 1e-5) * g + b


def _tc_encoder(xf, p, in_wh, in_wt, in_b, wm0, t, tw, tb, brows):
    nrows, din = xf.shape
    half = H // 2
    freq = jnp.exp(jnp.arange(half, dtype=jnp.float32)
                   * (-np.log(10000.0) / (half - 1)))[None, :]

    def body(x_ref, w1, b1, g1, bb1, w2, b2, g2, bb2, inw, inwt, inb, wm,
             t_ref, fr_ref, tw_ref, tb_ref, x0_ref, xw_ref):
        x = x_ref[...]
        h = _silu(_ln(x @ w1[...] + b1[...], g1[...], bb1[...]))
        h = _ln(h @ w2[...] + b2[...], g2[...], bb2[...])
        te_arg = t_ref[...] * fr_ref[...]
        te = (jnp.sum(jnp.sin(te_arg) * tw_ref[0:1, :])
              + jnp.sum(jnp.cos(te_arg) * tw_ref[1:2, :]) + tb_ref[0, 0])
        x0 = h @ inw[...] + te * inwt[...] + inb[...]
        x0_ref[...] = x0
        xw_ref[...] = x0 @ wm[...]

    full = lambda a, b: pl.BlockSpec((a, b), lambda i: (0, 0))
    return pl.pallas_call(
        body,
        grid=(nrows // brows,),
        in_specs=[
            pl.BlockSpec((brows, din), lambda i: (i, 0)),
            full(din, half), full(1, half), full(1, half), full(1, half),
            full(half, H), full(1, H), full(1, H), full(1, H),
            full(H, H), full(1, H), full(1, H), full(H, H),
            full(1, 1), full(1, half), full(2, half), full(1, 1),
        ],
        out_specs=[
            pl.BlockSpec((brows, H), lambda i: (i, 0)),
            pl.BlockSpec((brows, H), lambda i: (i, 0)),
        ],
        out_shape=[
            jax.ShapeDtypeStruct((nrows, H), jnp.float32),
            jax.ShapeDtypeStruct((nrows, H), jnp.float32),
        ],
    )(xf, p['w1'], p['b1'][None], p['g1'][None], p['bb1'][None],
      p['w2'], p['b2'][None], p['g2'][None], p['bb2'][None],
      in_wh, in_wt, in_b, wm0, t[:, None], freq, tw, tb)


def _sc_pos_diff(pos128, src, dst):
    @functools.partial(
        pl.kernel,
        mesh=_MESH,
        out_type=jax.ShapeDtypeStruct((E, 128), jnp.float32),
        scratch_types=[
            pltpu.VMEM((K,), jnp.int32),
            pltpu.VMEM((K,), jnp.int32),
            pltpu.VMEM((K, 128), jnp.float32),
            pltpu.VMEM((K, 128), jnp.float32),
            pltpu.SemaphoreType.DMA,
            pltpu.SemaphoreType.DMA,
        ],
    )
    def k(pos_hbm, src_hbm, dst_hbm, out_hbm, si_v, di_v, a_v, b_v, s1, s2):
        wid = lax.axis_index("s") * 2 + lax.axis_index("c")
        ept = E // NW
        nslab = ept // K

        def body(i, _):
            base = wid * ept + i * K
            pltpu.sync_copy(src_hbm.at[pl.ds(base, K)], si_v)
            pltpu.sync_copy(dst_hbm.at[pl.ds(base, K)], di_v)
            ca = pltpu.async_copy(pos_hbm.at[si_v], a_v, s1)
            cb = pltpu.async_copy(pos_hbm.at[di_v], b_v, s2)
            ca.wait()
            cb.wait()

            def row(r, _):
                b_v[r, pl.ds(0, 16)] = b_v[r, pl.ds(0, 16)] - a_v[r, pl.ds(0, 16)]
                return 0

            lax.fori_loop(0, K, row, 0, unroll=4)
            pltpu.sync_copy(b_v, out_hbm.at[pl.ds(base, K)])
            return 0

        lax.fori_loop(0, nslab, body, 0)

    return k(pos128, src, dst)


def _tc_edge_geom(diff16, etype, etype_emb):
    means = jnp.linspace(float(np.exp(-CUT)), 1.0, NRBF)[None, :]
    beta = float(((2.0 / NRBF) * (1.0 - np.exp(-CUT))) ** -2)

    def body(diff_ref, et_ref, emb_ref, means_ref, efb_ref, wts_ref):
        dif = diff_ref[...]
        dx, dy, dz = dif[:, 0:1], dif[:, 1:2], dif[:, 2:3]
        d = jnp.sqrt(dx * dx + dy * dy + dz * dz + 1e-12)
        ux, uy, uz = dx / d, dy / d, dz / d
        C = jnp.where(d < CUT, 0.5 * (jnp.cos(jnp.pi * d / CUT) + 1.0), 0.0)
        rbf = jnp.exp(-beta * (jnp.exp(-d) - means_ref[...]) ** 2)
        et = et_ref[...][:, 0]
        emb = emb_ref[...]
        emb_sel = (jnp.where((et == 0)[:, None], emb[0][None, :], 0.0)
                   + jnp.where((et == 1)[:, None], emb[1][None, :], 0.0)
                   + jnp.where((et == 2)[:, None], emb[2][None, :], 0.0))
        efb_ref[...] = rbf * C + emb_sel
        z = jnp.zeros_like(C)
        wts_ref[...] = jnp.concatenate([
            C, C * ux, C * uy, C * uz,
            C * ux * uy, C * uy * uz, C * uz * ux,
            C * (ux * ux - uy * uy), C * (3.0 * uz * uz - 1.0),
            z, z, z, z, z, z, z], axis=1)

    return pl.pallas_call(
        body,
        grid=(E // EB,),
        in_specs=[
            pl.BlockSpec((EB, 128), lambda i: (i, 0)),
            pl.BlockSpec((EB, 1), lambda i: (i, 0)),
            pl.BlockSpec((3, NRBF), lambda i: (0, 0)),
            pl.BlockSpec((1, NRBF), lambda i: (0, 0)),
        ],
        out_specs=[
            pl.BlockSpec((EB, NRBF), lambda i: (i, 0)),
            pl.BlockSpec((EB, 16), lambda i: (i, 0)),
        ],
        out_shape=[
            jax.ShapeDtypeStruct((E, NRBF), jnp.float32),
            jax.ShapeDtypeStruct((E, 16), jnp.float32),
        ],
    )(diff16, etype[:, None], etype_emb, means)


def _tc_edge_dense(efb, we, be, wm, bm):
    def body(efb_ref, we_r, be_r, wm_r, bm_r, out_ref):
        ef = _silu(efb_ref[...] @ we_r[...] + be_r[...])
        out_ref[...] = ef @ wm_r[...] + bm_r[...]

    return pl.pallas_call(
        body,
        grid=(E // EB,),
        in_specs=[
            pl.BlockSpec((EB, NRBF), lambda i: (i, 0)),
            pl.BlockSpec((NRBF, H), lambda i: (0, 0)),
            pl.BlockSpec((1, H), lambda i: (0, 0)),
            pl.BlockSpec((H, H), lambda i: (0, 0)),
            pl.BlockSpec((1, H), lambda i: (0, 0)),
        ],
        out_specs=pl.BlockSpec((EB, H), lambda i: (i, 0)),
        out_shape=jax.ShapeDtypeStruct((E, H), jnp.float32),
    )(efb, we, be[None], wm, bm[None])


def _sc_gather_silu(xw, efw, src):
    xw_pad = jnp.pad(xw, ((0, NPAD - xw.shape[0]), (0, 0)))
    KG = 120

    @functools.partial(
        pl.kernel,
        mesh=_MESH,
        out_type=jax.ShapeDtypeStruct((E, H), jnp.float32),
        scratch_types=[
            pltpu.VMEM((KG,), jnp.int32),
            pltpu.VMEM((KG, H), jnp.float32),
            pltpu.VMEM((KG, H), jnp.float32),
            pltpu.SemaphoreType.DMA,
            pltpu.VMEM_SHARED((NPAD, H), jnp.float32),
        ],
    )
    def k(xw_hbm, efw_hbm, src_hbm, out_hbm, idx_v, g_v, e_v, sem, xws):
        cid = lax.axis_index("c")
        sid = lax.axis_index("s")
        wid = sid * 2 + cid
        trows = NPAD // 16
        pltpu.sync_copy(xw_hbm.at[pl.ds(sid * trows, trows)],
                        xws.at[pl.ds(sid * trows, trows)])
        plsc.subcore_barrier()
        ept = E // NW
        nslab = ept // KG

        def body(i, _):
            base = wid * ept + i * KG
            pltpu.sync_copy(src_hbm.at[pl.ds(base, KG)], idx_v)
            ca = pltpu.async_copy(xws.at[idx_v], g_v, sem)
            pltpu.sync_copy(efw_hbm.at[pl.ds(base, KG)], e_v)
            ca.wait()

            def row(r, _):
                for j in range(H // 16):
                    p = g_v[r, pl.ds(j * 16, 16)] + e_v[r, pl.ds(j * 16, 16)]
                    e_v[r, pl.ds(j * 16, 16)] = p / (1.0 + jnp.exp(-p))
                return 0

            lax.fori_loop(0, KG, row, 0)
            pltpu.sync_copy(e_v, out_hbm.at[pl.ds(base, KG)])
            return 0

        lax.fori_loop(0, nslab, body, 0)

    return k(xw_pad, efw, src)


def _sc_edge_layer(xw, efw, wts_flat, src, dst, s_in):
    first = s_in is None
    xw_pad = jnp.pad(xw, ((0, NPAD - xw.shape[0]), (0, 0)))
    KS = 80

    @functools.partial(
        pl.kernel,
        mesh=_MESH,
        out_type=[
            jax.ShapeDtypeStruct((2 * NPAD, H), jnp.float32),
            jax.ShapeDtypeStruct((E, H), jnp.float32),
        ],
        scratch_types=[
            pltpu.VMEM((KS,), jnp.int32),
            pltpu.VMEM((KS,), jnp.int32),
            pltpu.VMEM((KS,), jnp.float32),
            pltpu.VMEM((KS, H), jnp.float32),
            pltpu.VMEM((KS, H), jnp.float32),
            pltpu.VMEM((16, H), jnp.float32),
            pltpu.SemaphoreType.DMA,
            pltpu.VMEM_SHARED((NPAD, H), jnp.float32),
        ],
    )
    def k(xw_hbm, efw_hbm, wts_hbm, src_hbm, dst_hbm, *rest):
        if first:
            s_hbm = None
            out_hbm, s_out = rest[:2]
            src_v, didx_v, w_v, g_v, e_v, z_v, sem, acc = rest[2:]
        else:
            s_hbm = rest[0]
            out_hbm, s_out = rest[1:3]
            src_v, didx_v, w_v, g_v, e_v, z_v, sem, acc = rest[3:]
        cid = lax.axis_index("c")
        sid = lax.axis_index("s")
        ept = E // NW
        nslab = ept // KS
        trows = NPAD // 16

        for r in range(16):
            for j in range(H // 16):
                z_v[r, pl.ds(j * 16, 16)] = jnp.zeros((16,), jnp.float32)

        def zbody(i, _):
            pltpu.sync_copy(z_v, acc.at[pl.ds(sid * trows + i * 16, 16)])
            return 0

        lax.fori_loop(0, trows // 16, zbody, 0)
        plsc.subcore_barrier()

        def body(i, _):
            base = cid * (E // 2) + sid * ept + i * KS
            pltpu.sync_copy(src_hbm.at[pl.ds(base, KS)], src_v)
            ca = pltpu.async_copy(xw_hbm.at[src_v], g_v, sem)
            pltpu.sync_copy(efw_hbm.at[pl.ds(base, KS)], e_v)
            pltpu.sync_copy(dst_hbm.at[pl.ds(base, KS)], didx_v)
            pltpu.sync_copy(wts_hbm.at[pl.ds(base, KS)], w_v)
            ca.wait()

            def row(r, _):
                for j in range(H // 16):
                    p = g_v[r, pl.ds(j * 16, 16)] + e_v[r, pl.ds(j * 16, 16)]
                    e_v[r, pl.ds(j * 16, 16)] = p / (1.0 + jnp.exp(-p))
                return 0

            lax.fori_loop(0, KS, row, 0)
            if first:
                pltpu.sync_copy(e_v, s_out.at[pl.ds(base, KS)])
            else:
                pltpu.sync_copy(s_hbm.at[pl.ds(base, KS)], g_v)

            def grp(g, _):
                w16 = w_v[pl.ds(g * 16, 16)]
                for l in range(16):
                    spl = w16.at[jnp.full((16,), l, jnp.int32)].get(
                        mode='promise_in_bounds')
                    for j in range(H // 16):
                        rr = g * 16 + l
                        m = e_v[rr, pl.ds(j * 16, 16)]
                        if not first:
                            g_v[rr, pl.ds(j * 16, 16)] = (
                                g_v[rr, pl.ds(j * 16, 16)] + m)
                        e_v[rr, pl.ds(j * 16, 16)] = m * spl
                return 0

            lax.fori_loop(0, KS // 16, grp, 0)
            if not first:
                pltpu.sync_copy(g_v, s_out.at[pl.ds(base, KS)])
            pltpu.sync_copy(e_v, acc.at[didx_v], add=True)
            return 0

        lax.fori_loop(0, nslab, body, 0)
        plsc.subcore_barrier()
        pltpu.sync_copy(acc.at[pl.ds(sid * trows, trows)],
                        out_hbm.at[pl.ds(cid * NPAD + sid * trows, trows)])

    if first:
        return k(xw_pad, efw, wts_flat, src, dst)
    return k(xw_pad, efw, wts_flat, src, dst, s_in)


def _sc_scatter0(sil, wts_flat, dst, s_in):
    first = s_in is None
    KS = 80
    scr = [
        pltpu.VMEM((KS,), jnp.int32),
        pltpu.VMEM((KS,), jnp.float32),
        pltpu.VMEM((KS, H), jnp.float32),
        pltpu.VMEM((KS, H), jnp.float32),
        pltpu.VMEM((16, H), jnp.float32),
        pltpu.VMEM_SHARED((NPAD, H), jnp.float32),
    ]

    @functools.partial(
        pl.kernel,
        mesh=_MESH,
        out_type=[
            jax.ShapeDtypeStruct((2 * NPAD, H), jnp.float32),
            jax.ShapeDtypeStruct((E, H), jnp.float32),
        ],
        scratch_types=scr,
    )
    def k(sil_hbm, wts_hbm, dst_hbm, *rest):
        if first:
            out_hbm, s_out = rest[:2]
            s_hbm = None
            didx_v, w_v, m_v, s_v, z_v, acc = rest[2:]
        else:
            s_hbm = rest[0]
            out_hbm, s_out = rest[1:3]
            didx_v, w_v, m_v, s_v, z_v, acc = rest[3:]
        cid = lax.axis_index("c")
        sid = lax.axis_index("s")
        ept = E // NW
        nslab = ept // KS
        trows = NPAD // 16

        for r in range(16):
            for j in range(H // 16):
                z_v[r, pl.ds(j * 16, 16)] = jnp.zeros((16,), jnp.float32)

        def zbody(i, _):
            pltpu.sync_copy(z_v, acc.at[pl.ds(sid * trows + i * 16, 16)])
            return 0

        lax.fori_loop(0, trows // 16, zbody, 0)
        plsc.subcore_barrier()

        def body(i, _):
            base = cid * (E // 2) + sid * ept + i * KS
            pltpu.sync_copy(dst_hbm.at[pl.ds(base, KS)], didx_v)
            pltpu.sync_copy(sil_hbm.at[pl.ds(base, KS)], m_v)
            pltpu.sync_copy(wts_hbm.at[pl.ds(base, KS)], w_v)
            if first:
                pltpu.sync_copy(m_v, s_out.at[pl.ds(base, KS)])
            else:
                pltpu.sync_copy(s_hbm.at[pl.ds(base, KS)], s_v)

            def grp(g, _):
                w16 = w_v[pl.ds(g * 16, 16)]
                for l in range(16):
                    spl = w16.at[jnp.full((16,), l, jnp.int32)].get(
                        mode='promise_in_bounds')
                    for j in range(H // 16):
                        rr = g * 16 + l
                        m = m_v[rr, pl.ds(j * 16, 16)]
                        if not first:
                            s_v[rr, pl.ds(j * 16, 16)] = (
                                s_v[rr, pl.ds(j * 16, 16)] + m)
                        m_v[rr, pl.ds(j * 16, 16)] = m * spl
                return 0

            lax.fori_loop(0, KS // 16, grp, 0)
            if not first:
                pltpu.sync_copy(s_v, s_out.at[pl.ds(base, KS)])
            pltpu.sync_copy(m_v, acc.at[didx_v], add=True)
            return 0

        lax.fori_loop(0, nslab, body, 0)
        plsc.subcore_barrier()
        pltpu.sync_copy(acc.at[pl.ds(sid * trows, trows)],
                        out_hbm.at[pl.ds(cid * NPAD + sid * trows, trows)])

    if first:
        return k(sil, wts_flat, dst)
    return k(sil, wts_flat, dst, s_in)


def _sc_scatter8(s, wts_flat, dst):
    KS = 160

    @functools.partial(
        pl.kernel,
        mesh=_MESH,
        out_type=jax.ShapeDtypeStruct((8 * NPAD, H), jnp.float32),
        scratch_types=[
            pltpu.VMEM((KS,), jnp.int32),
            pltpu.VMEM((KS,), jnp.float32),
            pltpu.VMEM((KS, H), jnp.float32),
            pltpu.VMEM((16, H), jnp.float32),
            pltpu.VMEM_SHARED((NPAD, H), jnp.float32),
        ],
    )
    def k(sil_hbm, wts_hbm, dst_hbm, out_hbm, didx_v, w_v, m_v, z_v, acc):
        cid = lax.axis_index("c")
        sid = lax.axis_index("s")
        ept = E // 16
        nslab = ept // KS
        trows = NPAD // 16

        for r in range(16):
            for j in range(H // 16):
                z_v[r, pl.ds(j * 16, 16)] = jnp.zeros((16,), jnp.float32)

        def one_pass(p, _):
            ci = p * 2 + cid

            def zbody(i, _):
                pltpu.sync_copy(z_v, acc.at[pl.ds(sid * trows + i * 16, 16)])
                return 0

            lax.fori_loop(0, trows // 16, zbody, 0)
            plsc.subcore_barrier()

            def body(i, _):
                base = sid * ept + i * KS
                pltpu.sync_copy(dst_hbm.at[pl.ds(base, KS)], didx_v)
                pltpu.sync_copy(sil_hbm.at[pl.ds(base, KS)], m_v)
                pltpu.sync_copy(wts_hbm.at[pl.ds((ci + 1) * E + base, KS)], w_v)

                def grp(g, _):
                    w16 = w_v[pl.ds(g * 16, 16)]
                    for l in range(16):
                        spl = w16.at[jnp.full((16,), l, jnp.int32)].get(
                            mode='promise_in_bounds')
                        for j in range(H // 16):
                            m_v[g * 16 + l, pl.ds(j * 16, 16)] = (
                                m_v[g * 16 + l, pl.ds(j * 16, 16)] * spl)
                    return 0

                lax.fori_loop(0, KS // 16, grp, 0)
                pltpu.sync_copy(m_v, acc.at[didx_v], add=True)
                return 0

            lax.fori_loop(0, nslab, body, 0)
            plsc.subcore_barrier()
            pltpu.sync_copy(acc.at[pl.ds(sid * trows, trows)],
                            out_hbm.at[pl.ds(ci * NPAD + sid * trows, trows)])
            plsc.subcore_barrier()
            return 0

        lax.fori_loop(0, 4, one_pass, 0)

    return k(s, wts_flat, dst)


def _tc_node_update(x, dv0a, dv0b, wu, bu, wm_next):
    def body(x_ref, dva_ref, dvb_ref, wu_r, bu_r, wm_r, xo_ref, xw_ref):
        y = x_ref[...] + dva_ref[...] + dvb_ref[...]
        y = y + _silu(y @ wu_r[...] + bu_r[...])
        xo_ref[...] = y
        xw_ref[...] = y @ wm_r[...]

    return pl.pallas_call(
        body,
        grid=(N // NB,),
        in_specs=[
            pl.BlockSpec((NB, H), lambda i: (i, 0)),
            pl.BlockSpec((NB, H), lambda i: (i, 0)),
            pl.BlockSpec((NB, H), lambda i: (i, 0)),
            pl.BlockSpec((H, H), lambda i: (0, 0)),
            pl.BlockSpec((1, H), lambda i: (0, 0)),
            pl.BlockSpec((H, H), lambda i: (0, 0)),
        ],
        out_specs=[
            pl.BlockSpec((NB, H), lambda i: (i, 0)),
            pl.BlockSpec((NB, H), lambda i: (i, 0)),
        ],
        out_shape=[
            jax.ShapeDtypeStruct((N, H), jnp.float32),
            jax.ShapeDtypeStruct((N, H), jnp.float32),
        ],
    )(x, dv0a, dv0b, wu, bu[None], wm_next)


def _geb_blk(x, v, wv1, wv2, u1w, u1b, u2w, u2b, dout, scalar_act):
    nv, nb, din = v.shape
    vec1 = (v.reshape(nv * nb, din) @ wv1).reshape(nv, nb, din)
    vec1n = jnp.sqrt(jnp.sum(vec1 * vec1, axis=0) + 1e-12)
    vec2 = (v.reshape(nv * nb, din) @ wv2).reshape(nv, nb, wv2.shape[1])
    h = jnp.concatenate([x, vec1n], axis=-1)
    h = _silu(h @ u1w + u1b)
    h = h @ u2w + u2b
    xo, gate = h[:, :dout], h[:, dout:]
    vo = vec2 * gate[None, :, :]
    if scalar_act:
        xo = _silu(xo)
    return xo, vo


def _tc_head(x, dv8, cw, fw):
    def body(x_ref, d1,
             c_wv1, c_wv2, c_u1w, c_u1b, c_u2w, c_u2b,
             c2_wv1, c2_wv2, c2_u1w, c2_u1b, c2_u2w, c2_u2b,
             sn_w1, sn_b1, sn_w2, sn_b2, l1_w, l1_b,
             l2_w1, l2_b1, l2_w2, l2_b2, comb,
             f_wv1, f_wv2, f_u1w, f_u1b, f_u2w, f_u2b,
             f2_wv1, f2_wv2, f2_u1w, f2_u1b, f2_u2w, f2_u2b,
             vel_ref, xf_ref):
        x = x_ref[...]
        v = d1[...]
        fsel0 = lambda r: r[...][0]
        xc, vc = _geb_blk(x, v, c_wv1[...], c_wv2[...], c_u1w[...],
                          c_u1b[...], c_u2w[...], c_u2b[...], H // 2, True)
        xc, vc = _geb_blk(xc, vc, c2_wv1[...], c2_wv2[...], c2_u1w[...],
                          c2_u1b[...], c2_u2w[...], c2_u2b[...], H // 4, True)
        l1w = l1_w[...]
        l1v = jnp.concatenate(
            [jnp.sum(vc[kk] * l1w, axis=-1, keepdims=True) for kk in range(3)],
            axis=1) + l1_b[...]
        vl2 = vc[3:8]
        l2m = jnp.sqrt(jnp.sum(vl2 * vl2, axis=0) + 1e-12)
        l2mod = jnp.tanh(_silu(l2m @ l2_w1[...] + l2_b1[...]) @ l2_w2[...] + l2_b2[...])
        mag = jax.nn.sigmoid(_silu(xc @ sn_w1[...] + sn_b1[...]) @ sn_w2[...] + sn_b2[...])
        c0 = comb[0, 0]
        c1 = comb[0, 1]
        e0 = jnp.exp(c0 - jnp.maximum(c0, c1))
        e1 = jnp.exp(c1 - jnp.maximum(c0, c1))
        w0 = e0 / (e0 + e1)
        w1 = e1 / (e0 + e1)
        vel = mag * ((w0 + w1 * l2mod) * l1v)
        vel_ref[...] = jnp.concatenate([vel, jnp.zeros((vel.shape[0], 5), jnp.float32)], axis=1)
        xa, va = _geb_blk(x, v, fsel0(f_wv1), fsel0(f_wv2), fsel0(f_u1w),
                          fsel0(f_u1b), fsel0(f_u2w), fsel0(f_u2b), H // 2, True)
        xa, _ = _geb_blk(xa, va, fsel0(f2_wv1), fsel0(f2_wv2), fsel0(f2_u1w),
                         fsel0(f2_u1b), fsel0(f2_u2w), fsel0(f2_u2b), RNF, False)
        xf_ref[...] = xa

    full = lambda *s: pl.BlockSpec(s, lambda i: (0,) * len(s))
    fsel = lambda *s: pl.BlockSpec((1,) + s[1:],
                                   lambda i: (i // 10,) + (0,) * (len(s) - 1))
    dspec = pl.BlockSpec((NV, NB, H), lambda i: (0, i, 0))
    vel, xf = pl.pallas_call(
        body,
        grid=(N // NB,),
        in_specs=[
            pl.BlockSpec((NB, H), lambda i: (i, 0)),
            dspec,
            full(H, H), full(H, H // 2), full(2 * H, H), full(1, H),
            full(H, H), full(1, H),
            full(H // 2, H // 2), full(H // 2, H // 4), full(H, H // 2),
            full(1, H // 2), full(H // 2, H // 2), full(1, H // 2),
            full(H // 4, H // 8), full(1, H // 8), full(H // 8, 1), full(1, 1),
            full(1, H // 4), full(1, 1),
            full(H // 4, H // 8), full(1, H // 8), full(H // 8, 1), full(1, 1),
            full(1, 2),
            fsel(2, H, H), fsel(2, H, H // 2), fsel(2, 2 * H, H), fsel(2, 1, H),
            fsel(2, H, H), fsel(2, 1, H),
            fsel(2, H // 2, H // 2), fsel(2, H // 2, RNF), fsel(2, H, H // 2),
            fsel(2, 1, H // 2), fsel(2, H // 2, 2 * RNF), fsel(2, 1, 2 * RNF),
        ],
        out_specs=[
            pl.BlockSpec((NB, 8), lambda i: (i, 0)),
            pl.BlockSpec((NB, RNF), lambda i: (i, 0)),
        ],
        out_shape=[
            jax.ShapeDtypeStruct((N, 8), jnp.float32),
            jax.ShapeDtypeStruct((N, RNF), jnp.float32),
        ],
    )(x, dv8, *cw, *fw)
    return vel, xf


def _pad_cols(a, w):
    return jnp.pad(a, ((0, 0), (0, w - a.shape[1])))


def kernel(xh_atoms, xh_residues, t, mask_atoms, mask_residues, edge_index, edge_types, params):
    src = edge_index[0].astype(jnp.int32)
    dst = edge_index[1].astype(jnp.int32)
    etype = edge_types.astype(jnp.int32)
    lp = params['layers']
    in_w = params['in_w']
    tw = jnp.stack([params['time_w'][:H // 2, 0], params['time_w'][H // 2:, 0]])
    tb = params['time_b'][None]

    x0a, xwa = _tc_encoder(xh_atoms[:, 3:], params['atom_enc'], in_w[:H],
                           in_w[H:H + 1], params['in_b'][None], lp[0]['wm'],
                           t, tw, tb, 2000)
    x0r, xwr = _tc_encoder(xh_residues[:, 3:], params['res_enc'], in_w[:H],
                           in_w[H:H + 1], params['in_b'][None], lp[0]['wm'],
                           t, tw, tb, 2000)
    x = jnp.concatenate([x0a, x0r], axis=0)
    xw = jnp.concatenate([xwa, xwr], axis=0)

    pos128 = jnp.zeros((N, 128), jnp.float32).at[:, :3].set(
        jnp.concatenate([xh_atoms[:, :3], xh_residues[:, :3]], axis=0))
    diff = _sc_pos_diff(pos128, src, dst)
    efb, wts_e = _tc_edge_geom(diff, etype, params['etype_emb'])
    wts_flat = wts_e.T.reshape(-1)

    s = None
    for li in range(NL):
        p = lp[li]
        efw = _tc_edge_dense(efb, p['we'], p['be'], p['wm'], p['bm'])
        dv0p, s = _sc_edge_layer(xw, efw, wts_flat, src, dst, s)
        dv0p = dv0p.reshape(2, NPAD, H)
        wm_next = lp[li + 1]['wm'] if li + 1 < NL else p['wu']
        x, xw = _tc_node_update(x, dv0p[0, :N], dv0p[1, :N],
                                p['wu'], p['bu'], wm_next)
    dv8 = _sc_scatter8(s, wts_flat, dst).reshape(8, NPAD, H)[:, :N]

    cp = params['coord']
    cw = []
    for g, dout in ((cp['g1'], H // 2), (cp['g2'], H // 4)):
        cw += [g['wv1'], g['wv2'], g['u1w'], g['u1b'][None], g['u2w'], g['u2b'][None]]
    cw += [cp['sn_w1'], cp['sn_b1'][None], cp['sn_w2'], cp['sn_b2'][None],
           cp['l1_w'].T, cp['l1_b'][None],
           cp['l2_w1'], cp['l2_b1'][None], cp['l2_w2'], cp['l2_b2'][None],
           cp['comb'][None]]

    fa, fr = params['feat_a'], params['feat_r']
    fw = []
    for key in ('wv1', 'wv2', 'u1w', 'u1b', 'u2w', 'u2b'):
        a, r = fa['g1'][key], fr['g1'][key]
        if a.ndim == 1:
            a, r = a[None], r[None]
        fw.append(jnp.stack([a, r]))
    a2, r2 = fa['g2'], fr['g2']
    u2w_a = jnp.concatenate([_pad_cols(a2['u2w'][:, :ANF], RNF),
                             _pad_cols(a2['u2w'][:, ANF:], RNF)], axis=1)
    u2b_a = jnp.concatenate([jnp.pad(a2['u2b'][:ANF], (0, RNF - ANF)),
                             jnp.pad(a2['u2b'][ANF:], (0, RNF - ANF))])
    for key, aw, rw in (('wv1', a2['wv1'], r2['wv1']),
                        ('wv2', _pad_cols(a2['wv2'], RNF), r2['wv2']),
                        ('u1w', a2['u1w'], r2['u1w']),
                        ('u1b', a2['u1b'][None], r2['u1b'][None]),
                        ('u2w', u2w_a, r2['u2w']),
                        ('u2b', u2b_a[None], r2['u2b'][None])):
        fw.append(jnp.stack([aw, rw]))

    vel, xf = _tc_head(x, dv8, cw, fw)
    out_a = jnp.concatenate([vel[:NA, :3], xf[:NA, :ANF]], axis=-1)
    out_r = jnp.concatenate([vel[NA:, :3], xf[NA:, :RNF]], axis=-1)
    return (out_a, out_r)
